# baseline (speedup 1.0000x reference)
import functools
import numpy as np
import jax
import jax.numpy as jnp
from jax.experimental import pallas as pl
from jax.experimental.pallas import tpu as pltpu

B, S, D = 1, 2048, 768
H, HKV, DH = 12, 4, 64
E, TOPK = 8, 2
FF = 3072
NA = S * TOPK
BS = 128
NPAD = NA + E * BS
NBLK = NPAD // BS
SB = 256
KB = 256
EPS = 1e-6
SCALE = 1.0 / float(np.sqrt(DH))


def _rope_tables_np():
    inv = 1.0 / (10000.0 ** (np.arange(0, DH, 2, dtype=np.float64) / DH))
    t = np.arange(S, dtype=np.float64)
    freqs = np.outer(t, inv)
    emb = np.concatenate([freqs, freqs], axis=-1)
    return np.cos(emb).astype(np.float32), np.sin(emb).astype(np.float32)


def _rot_perm_np(width):
    n = width // DH
    P = np.zeros((width, width), dtype=np.float32)
    half = DH // 2
    for c in range(n):
        b = c * DH
        for i in range(half):
            P[b + half + i, b + i] = -1.0
            P[b + i, b + half + i] = 1.0
    return P

_COS_NP, _SIN_NP = _rope_tables_np()
_PQ_NP = _rot_perm_np(H * DH)
_PK_NP = _rot_perm_np(HKV * DH)


def _rms(x, eps=EPS):
    v = jnp.mean(x * x, axis=-1, keepdims=True)
    return x * jax.lax.rsqrt(v + eps)


def _router_body(h_ref, wg1_ref, wg2_ref, logits_ref, w2_ref, sel2_ref):
    x = h_ref[...]
    t = jnp.dot(x, wg1_ref[...], preferred_element_type=jnp.float32)
    logits = jnp.dot(t, wg2_ref[...], preferred_element_type=jnp.float32)
    logits_ref[...] = logits
    m = jnp.max(logits, axis=-1, keepdims=True)
    p = jnp.exp(logits - m)
    rw = p / jnp.sum(p, axis=-1, keepdims=True)
    iota = jax.lax.broadcasted_iota(jnp.int32, rw.shape, 1)
    m0 = jnp.max(rw, axis=-1, keepdims=True)
    sel0 = jnp.min(jnp.where(rw >= m0, iota, E), axis=-1, keepdims=True)
    rw2 = jnp.where(iota == sel0, -1.0, rw)
    m1 = jnp.max(rw2, axis=-1, keepdims=True)
    sel1 = jnp.min(jnp.where(rw2 >= m1, iota, E), axis=-1, keepdims=True)
    den = m0 + m1 + 1e-9
    w2_ref[...] = jnp.concatenate([m0 / den, m1 / den], axis=-1)
    sel2_ref[...] = jnp.concatenate([sel0, sel1], axis=-1)


def _router(h2d, Wg1, Wg2):
    return pl.pallas_call(
        _router_body,
        grid=(S // SB,),
        in_specs=[
            pl.BlockSpec((SB, D), lambda i: (i, 0)),
            pl.BlockSpec((D, D), lambda i: (0, 0)),
            pl.BlockSpec((D, E), lambda i: (0, 0)),
        ],
        out_specs=[
            pl.BlockSpec((SB, E), lambda i: (i, 0)),
            pl.BlockSpec((SB, TOPK), lambda i: (i, 0)),
            pl.BlockSpec((SB, TOPK), lambda i: (i, 0)),
        ],
        out_shape=[
            jax.ShapeDtypeStruct((S, E), jnp.float32),
            jax.ShapeDtypeStruct((S, TOPK), jnp.float32),
            jax.ShapeDtypeStruct((S, TOPK), jnp.int32),
        ],
    )(h2d, Wg1, Wg2)


def _kv_body(h_ref, ln1_ref, wk_ref, wv_ref, cos_ref, sin_ref, pk_ref,
             k_ref, v_ref):
    x = (_rms(h_ref[...]) * ln1_ref[0]).astype(jnp.bfloat16)
    k = jnp.dot(x, wk_ref[0], preferred_element_type=jnp.float32)
    k = k * cos_ref[...] + jnp.dot(
        k.astype(jnp.bfloat16), pk_ref[...],
        preferred_element_type=jnp.float32) * sin_ref[...]
    k_ref[0] = k.astype(jnp.bfloat16)
    v_ref[0] = jnp.dot(x, wv_ref[0],
                       preferred_element_type=jnp.float32).astype(jnp.bfloat16)


def _kv_dense(h2d, ln1, Wk, Wv, cos4, sin4, Pk):
    return pl.pallas_call(
        _kv_body,
        grid=(E, S // SB),
        in_specs=[
            pl.BlockSpec((SB, D), lambda e, s: (s, 0)),
            pl.BlockSpec((1, 1, D), lambda e, s: (e, 0, 0)),
            pl.BlockSpec((1, D, HKV * DH), lambda e, s: (e, 0, 0)),
            pl.BlockSpec((1, D, HKV * DH), lambda e, s: (e, 0, 0)),
            pl.BlockSpec((SB, HKV * DH), lambda e, s: (s, 0)),
            pl.BlockSpec((SB, HKV * DH), lambda e, s: (s, 0)),
            pl.BlockSpec((HKV * DH, HKV * DH), lambda e, s: (0, 0)),
        ],
        out_specs=[
            pl.BlockSpec((1, SB, HKV * DH), lambda e, s: (e, s, 0)),
            pl.BlockSpec((1, SB, HKV * DH), lambda e, s: (e, s, 0)),
        ],
        out_shape=[
            jax.ShapeDtypeStruct((E, S, HKV * DH), jnp.bfloat16),
            jax.ShapeDtypeStruct((E, S, HKV * DH), jnp.bfloat16),
        ],
    )(h2d, ln1, Wk, Wv, cos4, sin4, Pk)


def _attn_body(be_ref, hs_ref, cos_ref, sin_ref, pos_ref, ln1_ref,
               wq_ref, wo_ref, pq_ref, k_ref, v_ref, y1_ref, o_ref):
    hs = hs_ref[...]
    x = (_rms(hs) * ln1_ref[0]).astype(jnp.bfloat16)
    q = jnp.dot(x, wq_ref[0], preferred_element_type=jnp.float32)
    q = q * cos_ref[...] + jnp.dot(
        q.astype(jnp.bfloat16), pq_ref[...],
        preferred_element_type=jnp.float32) * sin_ref[...]
    q = (q * SCALE).astype(jnp.bfloat16)
    pos_c = pos_ref[:, :1]
    maxpos = jnp.max(pos_ref[...])
    nk = maxpos // KB + 1
    pos3 = jnp.concatenate([pos_c, pos_c, pos_c], axis=0)
    G = H // HKV
    for kv in range(HKV):
        q3 = jnp.concatenate(
            [q[:, (G * kv + j) * DH:(G * kv + j + 1) * DH]
             for j in range(G)], axis=0)

        def body(j, carry, kv=kv, q3=q3):
            m, l, acc = carry
            ks = k_ref[0, pl.ds(j * KB, KB), kv * DH:(kv + 1) * DH]
            vs = v_ref[0, pl.ds(j * KB, KB), kv * DH:(kv + 1) * DH]
            s = jax.lax.dot_general(q3, ks, (((1,), (1,)), ((), ())),
                                    preferred_element_type=jnp.float32)
            kio = jax.lax.broadcasted_iota(
                jnp.int32, (G * BS, KB), 1) + j * KB
            s = jnp.where(pos3 >= kio, s, -1e30)
            mc = jnp.max(s, axis=-1, keepdims=True)
            mn = jnp.maximum(m, mc)
            corr = jnp.exp(m - mn)
            p = jnp.exp(s - mn)
            l2 = l * corr + jnp.sum(p, axis=-1, keepdims=True)
            acc2 = acc * corr + jnp.dot(p.astype(jnp.bfloat16), vs,
                                        preferred_element_type=jnp.float32)
            return (mn, l2, acc2)

        m0 = jnp.full((G * BS, 1), -1e30, jnp.float32)
        l0 = jnp.zeros((G * BS, 1), jnp.float32)
        a0 = jnp.zeros((G * BS, DH), jnp.float32)
        m, l, acc = jax.lax.fori_loop(0, nk, body, (m0, l0, a0))
        o3 = acc / l
        for j in range(G):
            o_ref[:, (G * kv + j) * DH:(G * kv + j + 1) * DH] = (
                o3[j * BS:(j + 1) * BS])
    y1_ref[...] = hs + jnp.dot(o_ref[...].astype(jnp.bfloat16), wo_ref[0],
                               preferred_element_type=jnp.float32)


def _attn_sparse(block_expert, hs, cos_s, sin_s, pos_col, ln1, Wq, Wo, Pq,
                 Kc, Vc):
    grid_spec = pltpu.PrefetchScalarGridSpec(
        num_scalar_prefetch=1,
        grid=(NBLK,),
        in_specs=[
            pl.BlockSpec((BS, D), lambda b, be: (b, 0)),
            pl.BlockSpec((BS, D), lambda b, be: (b, 0)),
            pl.BlockSpec((BS, D), lambda b, be: (b, 0)),
            pl.BlockSpec((BS, 128), lambda b, be: (b, 0)),
            pl.BlockSpec((1, 1, D), lambda b, be: (be[b], 0, 0)),
            pl.BlockSpec((1, D, H * DH), lambda b, be: (be[b], 0, 0)),
            pl.BlockSpec((1, H * DH, D), lambda b, be: (be[b], 0, 0)),
            pl.BlockSpec((H * DH, H * DH), lambda b, be: (0, 0)),
            pl.BlockSpec((1, S, HKV * DH), lambda b, be: (be[b], 0, 0)),
            pl.BlockSpec((1, S, HKV * DH), lambda b, be: (be[b], 0, 0)),
        ],
        out_specs=pl.BlockSpec((BS, D), lambda b, be: (b, 0)),
        scratch_shapes=[pltpu.VMEM((BS, H * DH), jnp.float32)],
    )
    return pl.pallas_call(
        _attn_body,
        grid_spec=grid_spec,
        out_shape=jax.ShapeDtypeStruct((NPAD, D), jnp.float32),
    )(block_expert, hs, cos_s, sin_s, pos_col, ln1, Wq, Wo, Pq, Kc, Vc)


def _mlp_body(be_ref, y1_ref, ln2_ref, wg_ref, wu_ref, wd_ref, y2_ref):
    a = y1_ref[...]
    x2 = (_rms(a) * ln2_ref[0]).astype(jnp.bfloat16)
    g = jnp.dot(x2, wg_ref[0], preferred_element_type=jnp.float32)
    u = jnp.dot(x2, wu_ref[0], preferred_element_type=jnp.float32)
    act = ((g / (1.0 + jnp.exp(-g))) * u).astype(jnp.bfloat16)
    y2_ref[...] = a + jnp.dot(act, wd_ref[0],
                              preferred_element_type=jnp.float32)


def _mlp_sparse(block_expert, y1, ln2, Wgate, Wup, Wdown):
    grid_spec = pltpu.PrefetchScalarGridSpec(
        num_scalar_prefetch=1,
        grid=(NBLK,),
        in_specs=[
            pl.BlockSpec((BS, D), lambda b, be: (b, 0)),
            pl.BlockSpec((1, 1, D), lambda b, be: (be[b], 0, 0)),
            pl.BlockSpec((1, D, FF), lambda b, be: (be[b], 0, 0)),
            pl.BlockSpec((1, D, FF), lambda b, be: (be[b], 0, 0)),
            pl.BlockSpec((1, FF, D), lambda b, be: (be[b], 0, 0)),
        ],
        out_specs=pl.BlockSpec((BS, D), lambda b, be: (b, 0)),
    )
    return pl.pallas_call(
        _mlp_body,
        grid_spec=grid_spec,
        out_shape=jax.ShapeDtypeStruct((NPAD, D), jnp.float32),
    )(block_expert, y1, ln2, Wgate, Wup, Wdown)


def _combine_body(g0_ref, g1_ref, w0_ref, w1_ref, out_ref):
    out_ref[...] = (g0_ref[...] * w0_ref[:, :1]
                    + g1_ref[...] * w1_ref[:, :1])


def _combine(g0, g1, w0c, w1c):
    return pl.pallas_call(
        _combine_body,
        grid=(S // SB,),
        in_specs=[
            pl.BlockSpec((SB, D), lambda i: (i, 0)),
            pl.BlockSpec((SB, D), lambda i: (i, 0)),
            pl.BlockSpec((SB, 128), lambda i: (i, 0)),
            pl.BlockSpec((SB, 128), lambda i: (i, 0)),
        ],
        out_specs=pl.BlockSpec((SB, D), lambda i: (i, 0)),
        out_shape=jax.ShapeDtypeStruct((S, D), jnp.float32),
    )(g0, g1, w0c, w1c)


@jax.jit
def kernel(hidden_states, Wg1, Wg2, ln1, ln2, Wq, Wk, Wv, Wo,
           Wgate, Wup, Wdown, position_ids):
    h2d = hidden_states[0]
    cos_t = jnp.asarray(_COS_NP)
    sin_t = jnp.asarray(_SIN_NP)
    pos = position_ids[0]
    cos_s_tab = jnp.tile(cos_t, (1, H))[pos]
    sin_s_tab = jnp.tile(sin_t, (1, H))[pos]
    cos4 = jnp.tile(cos_t, (1, HKV))[pos]
    sin4 = jnp.tile(sin_t, (1, HKV))[pos]
    Pq = jnp.asarray(_PQ_NP)
    Pk = jnp.asarray(_PK_NP)

    router_logits, w2, sel2 = _router(h2d, Wg1, Wg2)

    flat_sel = sel2.reshape(-1)
    tok = jnp.arange(NA, dtype=jnp.int32) // TOPK
    ohi = (flat_sel[:, None] ==
           jnp.arange(E, dtype=jnp.int32)[None, :]).astype(jnp.int32)
    rank = jnp.take_along_axis(jnp.cumsum(ohi, axis=0) - ohi,
                               flat_sel[:, None], axis=1)[:, 0]
    counts = jnp.sum(ohi, axis=0)
    padded = ((counts + BS - 1) // BS) * BS
    cum_pad = jnp.cumsum(padded)
    pstart = cum_pad - padded
    dest = (pstart[flat_sel] + rank).astype(jnp.int32)
    tok_sorted = jnp.zeros((NPAD,), jnp.int32).at[dest].set(tok)
    block_expert = jnp.minimum(
        jnp.searchsorted(cum_pad,
                         jnp.arange(NBLK, dtype=jnp.int32) * BS,
                         side='right'),
        E - 1).astype(jnp.int32)

    hs = h2d[tok_sorted]
    pos_sorted = pos[tok_sorted]
    cos_s = cos_s_tab[pos_sorted]
    sin_s = sin_s_tab[pos_sorted]
    pos_col = jnp.broadcast_to(pos_sorted[:, None], (NPAD, 128))

    ln1r = ln1[:, None, :]
    ln2r = ln2[:, None, :]
    bf = jnp.bfloat16
    Kc, Vc = _kv_dense(h2d, ln1r, Wk.astype(bf), Wv.astype(bf), cos4, sin4,
                       Pk.astype(bf))
    y1 = _attn_sparse(block_expert, hs, cos_s, sin_s, pos_col, ln1r,
                      Wq.astype(bf), Wo.astype(bf), Pq.astype(bf), Kc, Vc)
    y2 = _mlp_sparse(block_expert, y1, ln2r, Wgate.astype(bf),
                     Wup.astype(bf), Wdown.astype(bf))

    dest2 = dest.reshape(S, TOPK)
    g0 = y2[dest2[:, 0]]
    g1 = y2[dest2[:, 1]]
    w0c = jnp.broadcast_to(w2[:, :1], (S, 128))
    w1c = jnp.broadcast_to(w2[:, 1:2], (S, 128))
    final = _combine(g0, g1, w0c, w1c)

    return final[None], router_logits[None]


# speedup vs baseline: 1.3862x; 1.3862x over previous
import functools
import numpy as np
import jax
import jax.numpy as jnp
from jax import lax
from jax.experimental import pallas as pl
from jax.experimental.pallas import tpu as pltpu
from jax.experimental.pallas import tpu_sc as plsc

B, S, D = 1, 2048, 768
H, HKV, DH = 12, 4, 64
E, TOPK = 8, 2
FF = 3072
NA = S * TOPK
BS = 128
NPAD = NA + E * BS
NBLK = NPAD // BS
SB = 256
EPS = 1e-6
SCALE = 1.0 / float(np.sqrt(DH))

NC, NW = 2, 32
TPW = S // NW
SPW = NPAD // NW


def _rope_tables_np():
    inv = 1.0 / (10000.0 ** (np.arange(0, DH, 2, dtype=np.float64) / DH))
    t = np.arange(S, dtype=np.float64)
    freqs = np.outer(t, inv)
    emb = np.concatenate([freqs, freqs], axis=-1)
    return np.cos(emb).astype(np.float32), np.sin(emb).astype(np.float32)


def _rot_perm_np(width):
    n = width // DH
    P = np.zeros((width, width), dtype=np.float32)
    half = DH // 2
    for c in range(n):
        b = c * DH
        for i in range(half):
            P[b + half + i, b + i] = -1.0
            P[b + i, b + half + i] = 1.0
    return P

_COS_NP, _SIN_NP = _rope_tables_np()
_COS4_NP = np.tile(_COS_NP, (1, HKV))
_SIN4_NP = np.tile(_SIN_NP, (1, HKV))
_PQ_NP = _rot_perm_np(H * DH)
_PK_NP = _rot_perm_np(HKV * DH)


def _rms(x, eps=EPS):
    v = jnp.mean(x * x, axis=-1, keepdims=True)
    return x * jax.lax.rsqrt(v + eps)


def _router_body(h_ref, wg1_ref, wg2_ref, logits_ref, w2_ref, sel2_ref,
                 rank2_ref, counts_ref, carry_ref):
    i = pl.program_id(0)

    @pl.when(i == 0)
    def _():
        carry_ref[...] = jnp.zeros((1, E), jnp.float32)

    x = h_ref[...]
    t = jnp.dot(x, wg1_ref[...], preferred_element_type=jnp.float32)
    logits = jnp.dot(t, wg2_ref[...], preferred_element_type=jnp.float32)
    logits_ref[...] = logits
    m = jnp.max(logits, axis=-1, keepdims=True)
    p = jnp.exp(logits - m)
    rw = p / jnp.sum(p, axis=-1, keepdims=True)
    iota = jax.lax.broadcasted_iota(jnp.int32, rw.shape, 1)
    m0 = jnp.max(rw, axis=-1, keepdims=True)
    sel0 = jnp.min(jnp.where(rw >= m0, iota, E), axis=-1, keepdims=True)
    oh0 = (iota == sel0).astype(jnp.float32)
    rw2 = jnp.where(iota == sel0, -1.0, rw)
    m1 = jnp.max(rw2, axis=-1, keepdims=True)
    sel1 = jnp.min(jnp.where(rw2 >= m1, iota, E), axis=-1, keepdims=True)
    oh1 = (iota == sel1).astype(jnp.float32)
    den = m0 + m1 + 1e-9
    w2_ref[...] = jnp.concatenate([m0 / den, m1 / den], axis=-1)
    sel2_ref[...] = jnp.concatenate([sel0, sel1], axis=-1)

    ri = jax.lax.broadcasted_iota(jnp.int32, (SB, SB), 0)
    ci = jax.lax.broadcasted_iota(jnp.int32, (SB, SB), 1)
    tri = (ci < ri).astype(jnp.float32)
    both = oh0 + oh1
    cex = jnp.dot(tri, both, preferred_element_type=jnp.float32)
    carry = carry_ref[...]
    r0 = jnp.sum((carry + cex) * oh0, axis=-1, keepdims=True)
    r1 = jnp.sum((carry + cex) * oh1, axis=-1, keepdims=True)
    rank2_ref[...] = jnp.concatenate([r0, r1], axis=-1).astype(jnp.int32)
    carry = carry + jnp.sum(both, axis=0, keepdims=True)
    carry_ref[...] = carry
    counts_ref[...] = carry


def _router(h2d, Wg1, Wg2):
    return pl.pallas_call(
        _router_body,
        grid=(S // SB,),
        in_specs=[
            pl.BlockSpec((SB, D), lambda i: (i, 0)),
            pl.BlockSpec((D, D), lambda i: (0, 0)),
            pl.BlockSpec((D, E), lambda i: (0, 0)),
        ],
        out_specs=[
            pl.BlockSpec((SB, E), lambda i: (i, 0)),
            pl.BlockSpec((SB, TOPK), lambda i: (i, 0)),
            pl.BlockSpec((SB, TOPK), lambda i: (i, 0)),
            pl.BlockSpec((SB, TOPK), lambda i: (i, 0)),
            pl.BlockSpec((1, E), lambda i: (0, 0)),
        ],
        out_shape=[
            jax.ShapeDtypeStruct((S, E), jnp.float32),
            jax.ShapeDtypeStruct((S, TOPK), jnp.float32),
            jax.ShapeDtypeStruct((S, TOPK), jnp.int32),
            jax.ShapeDtypeStruct((S, TOPK), jnp.int32),
            jax.ShapeDtypeStruct((1, E), jnp.float32),
        ],
        scratch_shapes=[pltpu.VMEM((1, E), jnp.float32)],
    )(h2d, Wg1, Wg2)


def _dest_body(sel2_ref, rank2_ref, pst_ref, dest2_ref):
    sel = sel2_ref[...]
    acc = rank2_ref[...]
    for e in range(E):
        acc = acc + jnp.where(sel == e, pst_ref[0, e], 0)
    dest2_ref[...] = acc


def _dest(sel2, rank2, pstart16):
    return pl.pallas_call(
        _dest_body,
        grid=(1,),
        in_specs=[
            pl.BlockSpec((S, TOPK), lambda i: (0, 0)),
            pl.BlockSpec((S, TOPK), lambda i: (0, 0)),
            pl.BlockSpec(memory_space=pltpu.SMEM),
        ],
        out_specs=pl.BlockSpec((S, TOPK), lambda i: (0, 0)),
        out_shape=jax.ShapeDtypeStruct((S, TOPK), jnp.int32),
    )(sel2, rank2, pstart16)


def _scatter_sc(dest_flat):
    mesh = plsc.VectorSubcoreMesh(core_axis_name="c", subcore_axis_name="s")
    APW = NA // NW

    @functools.partial(
        pl.kernel,
        out_type=jax.ShapeDtypeStruct((NPAD,), jnp.int32),
        mesh=mesh,
        scratch_types=[
            pltpu.VMEM((APW,), jnp.int32),
            pltpu.VMEM((APW,), jnp.int32),
            pltpu.SemaphoreType.DMA,
        ],
    )
    def k(dest_h, tok_h, idx_v, val_v, sem):
        wid = lax.axis_index("s") * NC + lax.axis_index("c")
        base = wid * APW
        pltpu.sync_copy(dest_h.at[pl.ds(base, APW)], idx_v)
        iota16 = jax.lax.broadcasted_iota(jnp.int32, (16,), 0)

        def vbody(i, _):
            ent = base + i * 16 + iota16
            val_v[pl.ds(i * 16, 16)] = lax.shift_right_logical(ent, 1)
            return 0

        lax.fori_loop(0, APW // 16, vbody, 0)
        pltpu.async_copy(val_v, tok_h.at[idx_v], sem).wait()

    return k(dest_flat)


_GC = 5
_GR = SPW // _GC


def _gather_sc(tok_sorted, h2d, cossin):
    mesh = plsc.VectorSubcoreMesh(core_axis_name="c", subcore_axis_name="s")

    @functools.partial(
        pl.kernel,
        out_type=[
            jax.ShapeDtypeStruct((NPAD, D), jnp.float32),
            jax.ShapeDtypeStruct((NPAD, 2 * DH), jnp.float32),
        ],
        mesh=mesh,
        scratch_types=[
            pltpu.VMEM((_GR,), jnp.int32),
            pltpu.VMEM((_GR, D), jnp.float32),
            pltpu.VMEM((_GR, 2 * DH), jnp.float32),
            pltpu.SemaphoreType.DMA,
        ],
    )
    def k(tok_h, h_h, cs_h, hs_h, csg_h, idx_v, hbuf, cbuf, sem):
        wid = lax.axis_index("s") * NC + lax.axis_index("c")
        base = wid * SPW
        for c in range(_GC):
            rs = pl.ds(base + c * _GR, _GR)
            pltpu.sync_copy(tok_h.at[rs], idx_v)

            def clamp(i, _):
                sl = pl.ds(i * 16, 16)
                idx_v[sl] = jnp.minimum(
                    jnp.maximum(idx_v[sl], 0), S - 1)
                return 0

            lax.fori_loop(0, _GR // 16, clamp, 0)
            pltpu.async_copy(h_h.at[idx_v], hbuf, sem).wait()
            pltpu.sync_copy(hbuf, hs_h.at[rs])
            pltpu.async_copy(cs_h.at[idx_v], cbuf, sem).wait()
            pltpu.sync_copy(cbuf, csg_h.at[rs])

    return k(tok_sorted, h2d, cossin)


_CC = 2
_CR = TPW // _CC


def _combine_gather_sc(y2, d0, d1):
    mesh = plsc.VectorSubcoreMesh(core_axis_name="c", subcore_axis_name="s")

    @functools.partial(
        pl.kernel,
        out_type=[
            jax.ShapeDtypeStruct((S, D), jnp.float32),
            jax.ShapeDtypeStruct((S, D), jnp.float32),
        ],
        mesh=mesh,
        scratch_types=[
            pltpu.VMEM((_CR,), jnp.int32),
            pltpu.VMEM((_CR, D), jnp.float32),
            pltpu.SemaphoreType.DMA,
        ],
    )
    def k(y2_h, d0_h, d1_h, g0_h, g1_h, idx_v, buf, sem):
        wid = lax.axis_index("s") * NC + lax.axis_index("c")
        base = wid * TPW
        for c in range(_CC):
            rs = pl.ds(base + c * _CR, _CR)
            pltpu.sync_copy(d0_h.at[rs], idx_v)
            pltpu.async_copy(y2_h.at[idx_v], buf, sem).wait()
            pltpu.sync_copy(buf, g0_h.at[rs])
            pltpu.sync_copy(d1_h.at[rs], idx_v)
            pltpu.async_copy(y2_h.at[idx_v], buf, sem).wait()
            pltpu.sync_copy(buf, g1_h.at[rs])

    return k(y2, d0, d1)


def _kv_body(h_ref, ln1_ref, wk_ref, wv_ref, cos_ref, sin_ref, pk_ref,
             k_ref, v_ref):
    x = _rms(h_ref[...]) * ln1_ref[0]
    k = jnp.dot(x, wk_ref[0], preferred_element_type=jnp.float32)
    k = k * cos_ref[...] + jnp.dot(
        k, pk_ref[...], preferred_element_type=jnp.float32) * sin_ref[...]
    k_ref[0] = k
    v_ref[0] = jnp.dot(x, wv_ref[0], preferred_element_type=jnp.float32)


def _kv_dense(h2d, ln1, Wk, Wv, cos4, sin4, Pk):
    return pl.pallas_call(
        _kv_body,
        grid=(E, S // SB),
        in_specs=[
            pl.BlockSpec((SB, D), lambda e, s: (s, 0)),
            pl.BlockSpec((1, 1, D), lambda e, s: (e, 0, 0)),
            pl.BlockSpec((1, D, HKV * DH), lambda e, s: (e, 0, 0)),
            pl.BlockSpec((1, D, HKV * DH), lambda e, s: (e, 0, 0)),
            pl.BlockSpec((SB, HKV * DH), lambda e, s: (s, 0)),
            pl.BlockSpec((SB, HKV * DH), lambda e, s: (s, 0)),
            pl.BlockSpec((HKV * DH, HKV * DH), lambda e, s: (0, 0)),
        ],
        out_specs=[
            pl.BlockSpec((1, SB, HKV * DH), lambda e, s: (e, s, 0)),
            pl.BlockSpec((1, SB, HKV * DH), lambda e, s: (e, s, 0)),
        ],
        out_shape=[
            jax.ShapeDtypeStruct((E, S, HKV * DH), jnp.float32),
            jax.ShapeDtypeStruct((E, S, HKV * DH), jnp.float32),
        ],
    )(h2d, ln1, Wk, Wv, cos4, sin4, Pk)


def _attn_body(be_ref, hs_ref, cs_ref, pos_ref, ln1_ref,
               wq_ref, wo_ref, pq_ref, k_ref, v_ref, y1_ref, o_ref):
    hs = hs_ref[...]
    x = _rms(hs) * ln1_ref[0]
    q = jnp.dot(x, wq_ref[0], preferred_element_type=jnp.float32)
    cfull = jnp.concatenate([cs_ref[:, :DH]] * H, axis=1)
    sfull = jnp.concatenate([cs_ref[:, DH:]] * H, axis=1)
    q = q * cfull + jnp.dot(
        q, pq_ref[...], preferred_element_type=jnp.float32) * sfull
    q = q * SCALE
    pos_q = pos_ref[...]
    kiota = jax.lax.broadcasted_iota(jnp.int32, (BS, S), 1)
    mask = pos_q[:, :1] >= kiota
    for hh in range(H):
        kv = hh // (H // HKV)
        qh = q[:, hh * DH:(hh + 1) * DH]
        kh = k_ref[0][:, kv * DH:(kv + 1) * DH]
        vh = v_ref[0][:, kv * DH:(kv + 1) * DH]
        s = jax.lax.dot_general(qh, kh, (((1,), (1,)), ((), ())),
                                preferred_element_type=jnp.float32)
        s = jnp.where(mask, s, -1e30)
        m = jnp.max(s, axis=-1, keepdims=True)
        p = jnp.exp(s - m)
        p = p / jnp.sum(p, axis=-1, keepdims=True)
        o_ref[:, hh * DH:(hh + 1) * DH] = jnp.dot(
            p, vh, preferred_element_type=jnp.float32)
    y1_ref[...] = hs + jnp.dot(o_ref[...], wo_ref[0],
                               preferred_element_type=jnp.float32)


def _attn_sparse(block_expert, hs, cs_g, pos_col, ln1, Wq, Wo, Pq,
                 Kc, Vc):
    grid_spec = pltpu.PrefetchScalarGridSpec(
        num_scalar_prefetch=1,
        grid=(NBLK,),
        in_specs=[
            pl.BlockSpec((BS, D), lambda b, be: (b, 0)),
            pl.BlockSpec((BS, 2 * DH), lambda b, be: (b, 0)),
            pl.BlockSpec((BS, 128), lambda b, be: (b, 0)),
            pl.BlockSpec((1, 1, D), lambda b, be: (be[b], 0, 0)),
            pl.BlockSpec((1, D, H * DH), lambda b, be: (be[b], 0, 0)),
            pl.BlockSpec((1, H * DH, D), lambda b, be: (be[b], 0, 0)),
            pl.BlockSpec((H * DH, H * DH), lambda b, be: (0, 0)),
            pl.BlockSpec((1, S, HKV * DH), lambda b, be: (be[b], 0, 0)),
            pl.BlockSpec((1, S, HKV * DH), lambda b, be: (be[b], 0, 0)),
        ],
        out_specs=pl.BlockSpec((BS, D), lambda b, be: (b, 0)),
        scratch_shapes=[pltpu.VMEM((BS, H * DH), jnp.float32)],
    )
    return pl.pallas_call(
        _attn_body,
        grid_spec=grid_spec,
        out_shape=jax.ShapeDtypeStruct((NPAD, D), jnp.float32),
    )(block_expert, hs, cs_g, pos_col, ln1, Wq, Wo, Pq, Kc, Vc)


def _mlp_body(be_ref, y1_ref, ln2_ref, wg_ref, wu_ref, wd_ref, y2_ref):
    a = y1_ref[...]
    x2 = _rms(a) * ln2_ref[0]
    g = jnp.dot(x2, wg_ref[0], preferred_element_type=jnp.float32)
    u = jnp.dot(x2, wu_ref[0], preferred_element_type=jnp.float32)
    act = (g / (1.0 + jnp.exp(-g))) * u
    y2_ref[...] = a + jnp.dot(act, wd_ref[0],
                              preferred_element_type=jnp.float32)


def _mlp_sparse(block_expert, y1, ln2, Wgate, Wup, Wdown):
    grid_spec = pltpu.PrefetchScalarGridSpec(
        num_scalar_prefetch=1,
        grid=(NBLK,),
        in_specs=[
            pl.BlockSpec((BS, D), lambda b, be: (b, 0)),
            pl.BlockSpec((1, 1, D), lambda b, be: (be[b], 0, 0)),
            pl.BlockSpec((1, D, FF), lambda b, be: (be[b], 0, 0)),
            pl.BlockSpec((1, D, FF), lambda b, be: (be[b], 0, 0)),
            pl.BlockSpec((1, FF, D), lambda b, be: (be[b], 0, 0)),
        ],
        out_specs=pl.BlockSpec((BS, D), lambda b, be: (b, 0)),
    )
    return pl.pallas_call(
        _mlp_body,
        grid_spec=grid_spec,
        out_shape=jax.ShapeDtypeStruct((NPAD, D), jnp.float32),
    )(block_expert, y1, ln2, Wgate, Wup, Wdown)


def _combine_body(g0_ref, g1_ref, w0_ref, w1_ref, out_ref):
    out_ref[...] = (g0_ref[...] * w0_ref[:, :1]
                    + g1_ref[...] * w1_ref[:, :1])


def _combine(g0, g1, w0c, w1c):
    return pl.pallas_call(
        _combine_body,
        grid=(S // SB,),
        in_specs=[
            pl.BlockSpec((SB, D), lambda i: (i, 0)),
            pl.BlockSpec((SB, D), lambda i: (i, 0)),
            pl.BlockSpec((SB, 128), lambda i: (i, 0)),
            pl.BlockSpec((SB, 128), lambda i: (i, 0)),
        ],
        out_specs=pl.BlockSpec((SB, D), lambda i: (i, 0)),
        out_shape=jax.ShapeDtypeStruct((S, D), jnp.float32),
    )(g0, g1, w0c, w1c)


@jax.jit
def kernel(hidden_states, Wg1, Wg2, ln1, ln2, Wq, Wk, Wv, Wo,
           Wgate, Wup, Wdown, position_ids):
    h2d = hidden_states[0]
    cos_t = jnp.asarray(_COS_NP)
    sin_t = jnp.asarray(_SIN_NP)
    cos4 = jnp.asarray(_COS4_NP)
    sin4 = jnp.asarray(_SIN4_NP)
    Pq = jnp.asarray(_PQ_NP)
    Pk = jnp.asarray(_PK_NP)

    router_logits, w2, sel2, rank2, countsf = _router(h2d, Wg1, Wg2)

    counts = countsf[0].astype(jnp.int32)
    padded = ((counts + BS - 1) // BS) * BS
    cum_pad = jnp.cumsum(padded)
    pstart = (cum_pad - padded).astype(jnp.int32)
    block_expert = jnp.minimum(
        jnp.searchsorted(cum_pad,
                         jnp.arange(NBLK, dtype=jnp.int32) * BS,
                         side='right'),
        E - 1).astype(jnp.int32)

    pstart16 = jnp.concatenate(
        [pstart, jnp.zeros((16 - E,), jnp.int32)])[None, :]
    dest2 = _dest(sel2, rank2, pstart16)
    d0 = dest2[:, 0]
    d1 = dest2[:, 1]
    tok_sorted = _scatter_sc(dest2.reshape(-1))
    cossin = jnp.concatenate([cos_t, sin_t], axis=1)
    hs, cs_g = _gather_sc(tok_sorted, h2d, cossin)
    pos_col = jnp.broadcast_to(tok_sorted[:, None], (NPAD, 128))

    Kc, Vc = _kv_dense(h2d, ln1[:, None, :], Wk, Wv, cos4, sin4, Pk)
    y1 = _attn_sparse(block_expert, hs, cs_g, pos_col,
                      ln1[:, None, :], Wq, Wo, Pq, Kc, Vc)
    y2 = _mlp_sparse(block_expert, y1, ln2[:, None, :], Wgate, Wup, Wdown)

    g0, g1 = _combine_gather_sc(y2, d0, d1)
    w0c = jnp.broadcast_to(w2[:, :1], (S, 128))
    w1c = jnp.broadcast_to(w2[:, 1:2], (S, 128))
    final = _combine(g0, g1, w0c, w1c)

    return final[None], router_logits[None]


# speedup vs baseline: 1.4254x; 1.0283x over previous
import functools
import numpy as np
import jax
import jax.numpy as jnp
from jax import lax
from jax.experimental import pallas as pl
from jax.experimental.pallas import tpu as pltpu
from jax.experimental.pallas import tpu_sc as plsc

B, S, D = 1, 2048, 768
H, HKV, DH = 12, 4, 64
E, TOPK = 8, 2
FF = 3072
NA = S * TOPK
BS = 128
NPAD = NA + E * BS
NBLK = NPAD // BS
SB = 256
EPS = 1e-6
SCALE = 1.0 / float(np.sqrt(DH))

NC, NW = 2, 32
TPW = S // NW
SPW = NPAD // NW


def _rope_tables_np():
    inv = 1.0 / (10000.0 ** (np.arange(0, DH, 2, dtype=np.float64) / DH))
    t = np.arange(S, dtype=np.float64)
    freqs = np.outer(t, inv)
    emb = np.concatenate([freqs, freqs], axis=-1)
    return np.cos(emb).astype(np.float32), np.sin(emb).astype(np.float32)


def _rot_perm_np(width):
    n = width // DH
    P = np.zeros((width, width), dtype=np.float32)
    half = DH // 2
    for c in range(n):
        b = c * DH
        for i in range(half):
            P[b + half + i, b + i] = -1.0
            P[b + i, b + half + i] = 1.0
    return P

_COS_NP, _SIN_NP = _rope_tables_np()
_COS4_NP = np.tile(_COS_NP, (1, HKV))
_SIN4_NP = np.tile(_SIN_NP, (1, HKV))
_PQ_NP = _rot_perm_np(H * DH)
_PK_NP = _rot_perm_np(HKV * DH)


def _rms(x, eps=EPS):
    v = jnp.mean(x * x, axis=-1, keepdims=True)
    return x * jax.lax.rsqrt(v + eps)


def _router_body(h_ref, wg1_ref, wg2_ref, logits_ref, w2_ref, sel2_ref,
                 rank2_ref, counts_ref, carry_ref):
    i = pl.program_id(0)

    @pl.when(i == 0)
    def _():
        carry_ref[...] = jnp.zeros((1, E), jnp.float32)

    x = h_ref[...]
    t = jnp.dot(x, wg1_ref[...], preferred_element_type=jnp.float32)
    logits = jnp.dot(t, wg2_ref[...], preferred_element_type=jnp.float32)
    logits_ref[...] = logits
    m = jnp.max(logits, axis=-1, keepdims=True)
    p = jnp.exp(logits - m)
    rw = p / jnp.sum(p, axis=-1, keepdims=True)
    iota = jax.lax.broadcasted_iota(jnp.int32, rw.shape, 1)
    m0 = jnp.max(rw, axis=-1, keepdims=True)
    sel0 = jnp.min(jnp.where(rw >= m0, iota, E), axis=-1, keepdims=True)
    oh0 = (iota == sel0).astype(jnp.float32)
    rw2 = jnp.where(iota == sel0, -1.0, rw)
    m1 = jnp.max(rw2, axis=-1, keepdims=True)
    sel1 = jnp.min(jnp.where(rw2 >= m1, iota, E), axis=-1, keepdims=True)
    oh1 = (iota == sel1).astype(jnp.float32)
    den = m0 + m1 + 1e-9
    w2_ref[...] = jnp.concatenate([m0 / den, m1 / den], axis=-1)
    sel2_ref[...] = jnp.concatenate([sel0, sel1], axis=-1)

    ri = jax.lax.broadcasted_iota(jnp.int32, (SB, SB), 0)
    ci = jax.lax.broadcasted_iota(jnp.int32, (SB, SB), 1)
    tri = (ci < ri).astype(jnp.float32)
    both = oh0 + oh1
    cex = jnp.dot(tri, both, preferred_element_type=jnp.float32)
    carry = carry_ref[...]
    r0 = jnp.sum((carry + cex) * oh0, axis=-1, keepdims=True)
    r1 = jnp.sum((carry + cex) * oh1, axis=-1, keepdims=True)
    rank2_ref[...] = jnp.concatenate([r0, r1], axis=-1).astype(jnp.int32)
    carry = carry + jnp.sum(both, axis=0, keepdims=True)
    carry_ref[...] = carry
    counts_ref[...] = carry


def _router(h2d, Wg1, Wg2):
    return pl.pallas_call(
        _router_body,
        grid=(S // SB,),
        in_specs=[
            pl.BlockSpec((SB, D), lambda i: (i, 0)),
            pl.BlockSpec((D, D), lambda i: (0, 0)),
            pl.BlockSpec((D, E), lambda i: (0, 0)),
        ],
        out_specs=[
            pl.BlockSpec((SB, E), lambda i: (i, 0)),
            pl.BlockSpec((SB, TOPK), lambda i: (i, 0)),
            pl.BlockSpec((SB, TOPK), lambda i: (i, 0)),
            pl.BlockSpec((SB, TOPK), lambda i: (i, 0)),
            pl.BlockSpec((1, E), lambda i: (0, 0)),
        ],
        out_shape=[
            jax.ShapeDtypeStruct((S, E), jnp.float32),
            jax.ShapeDtypeStruct((S, TOPK), jnp.float32),
            jax.ShapeDtypeStruct((S, TOPK), jnp.int32),
            jax.ShapeDtypeStruct((S, TOPK), jnp.int32),
            jax.ShapeDtypeStruct((1, E), jnp.float32),
        ],
        scratch_shapes=[pltpu.VMEM((1, E), jnp.float32)],
    )(h2d, Wg1, Wg2)


def _dest_body(sel2_ref, rank2_ref, pst_ref, dest2_ref):
    sel = sel2_ref[...]
    acc = rank2_ref[...]
    for e in range(E):
        acc = acc + jnp.where(sel == e, pst_ref[0, e], 0)
    dest2_ref[...] = acc


def _dest(sel2, rank2, pstart16):
    return pl.pallas_call(
        _dest_body,
        grid=(1,),
        in_specs=[
            pl.BlockSpec((S, TOPK), lambda i: (0, 0)),
            pl.BlockSpec((S, TOPK), lambda i: (0, 0)),
            pl.BlockSpec(memory_space=pltpu.SMEM),
        ],
        out_specs=pl.BlockSpec((S, TOPK), lambda i: (0, 0)),
        out_shape=jax.ShapeDtypeStruct((S, TOPK), jnp.int32),
    )(sel2, rank2, pstart16)


def _scatter_sc(dest_flat):
    mesh = plsc.VectorSubcoreMesh(core_axis_name="c", subcore_axis_name="s")
    APW = NA // NW

    @functools.partial(
        pl.kernel,
        out_type=jax.ShapeDtypeStruct((NPAD,), jnp.int32),
        mesh=mesh,
        scratch_types=[
            pltpu.VMEM((APW,), jnp.int32),
            pltpu.VMEM((APW,), jnp.int32),
            pltpu.SemaphoreType.DMA,
        ],
    )
    def k(dest_h, tok_h, idx_v, val_v, sem):
        wid = lax.axis_index("s") * NC + lax.axis_index("c")
        base = wid * APW
        pltpu.sync_copy(dest_h.at[pl.ds(base, APW)], idx_v)
        iota16 = jax.lax.broadcasted_iota(jnp.int32, (16,), 0)

        def vbody(i, _):
            ent = base + i * 16 + iota16
            val_v[pl.ds(i * 16, 16)] = lax.shift_right_logical(ent, 1)
            return 0

        lax.fori_loop(0, APW // 16, vbody, 0)
        pltpu.async_copy(val_v, tok_h.at[idx_v], sem).wait()

    return k(dest_flat)


_GC = 5
_GR = SPW // _GC


def _gather_sc(tok_sorted, h2d, cossin):
    mesh = plsc.VectorSubcoreMesh(core_axis_name="c", subcore_axis_name="s")

    @functools.partial(
        pl.kernel,
        out_type=[
            jax.ShapeDtypeStruct((NPAD, D), jnp.float32),
            jax.ShapeDtypeStruct((NPAD, 2 * DH), jnp.float32),
        ],
        mesh=mesh,
        scratch_types=[
            pltpu.VMEM((2, _GR), jnp.int32),
            pltpu.VMEM((2, _GR, D), jnp.float32),
            pltpu.VMEM((2, _GR, 2 * DH), jnp.float32),
            pltpu.SemaphoreType.DMA,
            pltpu.SemaphoreType.DMA,
        ],
    )
    def k(tok_h, h_h, cs_h, hs_h, csg_h, idx_v, hbuf, cbuf, sem0, sem1):
        wid = lax.axis_index("s") * NC + lax.axis_index("c")
        base = wid * SPW
        sems = (sem0, sem1)

        def start(c):
            b = c & 1
            pltpu.sync_copy(tok_h.at[pl.ds(base + c * _GR, _GR)],
                            idx_v.at[b])

            def clamp(i, _):
                sl = pl.ds(i * 16, 16)
                idx_v[b, sl] = jnp.minimum(
                    jnp.maximum(idx_v[b, sl], 0), S - 1)
                return 0

            lax.fori_loop(0, _GR // 16, clamp, 0)
            hcp = pltpu.async_copy(h_h.at[idx_v.at[b]], hbuf.at[b], sems[b])
            ccp = pltpu.async_copy(cs_h.at[idx_v.at[b]], cbuf.at[b], sems[b])
            return hcp, ccp

        def drain(c, cps):
            b = c & 1
            rs = pl.ds(base + c * _GR, _GR)
            cps[0].wait()
            cps[1].wait()
            pltpu.sync_copy(hbuf.at[b], hs_h.at[rs])
            pltpu.sync_copy(cbuf.at[b], csg_h.at[rs])

        prev = start(0)
        for c in range(1, _GC):
            cur = start(c)
            drain(c - 1, prev)
            prev = cur
        drain(_GC - 1, prev)

    return k(tok_sorted, h2d, cossin)


_CC = 2
_CR = TPW // _CC


def _combine_gather_sc(y2, d0, d1):
    mesh = plsc.VectorSubcoreMesh(core_axis_name="c", subcore_axis_name="s")

    @functools.partial(
        pl.kernel,
        out_type=[
            jax.ShapeDtypeStruct((S, D), jnp.float32),
            jax.ShapeDtypeStruct((S, D), jnp.float32),
        ],
        mesh=mesh,
        scratch_types=[
            pltpu.VMEM((_CR,), jnp.int32),
            pltpu.VMEM((_CR, D), jnp.float32),
            pltpu.SemaphoreType.DMA,
        ],
    )
    def k(y2_h, d0_h, d1_h, g0_h, g1_h, idx_v, buf, sem):
        wid = lax.axis_index("s") * NC + lax.axis_index("c")
        base = wid * TPW
        for c in range(_CC):
            rs = pl.ds(base + c * _CR, _CR)
            pltpu.sync_copy(d0_h.at[rs], idx_v)
            pltpu.async_copy(y2_h.at[idx_v], buf, sem).wait()
            pltpu.sync_copy(buf, g0_h.at[rs])
            pltpu.sync_copy(d1_h.at[rs], idx_v)
            pltpu.async_copy(y2_h.at[idx_v], buf, sem).wait()
            pltpu.sync_copy(buf, g1_h.at[rs])

    return k(y2, d0, d1)


def _kv_body(h_ref, ln1_ref, wk_ref, wv_ref, cos_ref, sin_ref, pk_ref,
             k_ref, v_ref):
    x = _rms(h_ref[...]) * ln1_ref[0]
    k = jnp.dot(x, wk_ref[0], preferred_element_type=jnp.float32)
    k = k * cos_ref[...] + jnp.dot(
        k, pk_ref[...], preferred_element_type=jnp.float32) * sin_ref[...]
    k_ref[0] = k
    v_ref[0] = jnp.dot(x, wv_ref[0], preferred_element_type=jnp.float32)


def _kv_dense(h2d, ln1, Wk, Wv, cos4, sin4, Pk):
    return pl.pallas_call(
        _kv_body,
        grid=(E, S // SB),
        in_specs=[
            pl.BlockSpec((SB, D), lambda e, s: (s, 0)),
            pl.BlockSpec((1, 1, D), lambda e, s: (e, 0, 0)),
            pl.BlockSpec((1, D, HKV * DH), lambda e, s: (e, 0, 0)),
            pl.BlockSpec((1, D, HKV * DH), lambda e, s: (e, 0, 0)),
            pl.BlockSpec((SB, HKV * DH), lambda e, s: (s, 0)),
            pl.BlockSpec((SB, HKV * DH), lambda e, s: (s, 0)),
            pl.BlockSpec((HKV * DH, HKV * DH), lambda e, s: (0, 0)),
        ],
        out_specs=[
            pl.BlockSpec((1, SB, HKV * DH), lambda e, s: (e, s, 0)),
            pl.BlockSpec((1, SB, HKV * DH), lambda e, s: (e, s, 0)),
        ],
        out_shape=[
            jax.ShapeDtypeStruct((E, S, HKV * DH), jnp.float32),
            jax.ShapeDtypeStruct((E, S, HKV * DH), jnp.float32),
        ],
    )(h2d, ln1, Wk, Wv, cos4, sin4, Pk)


def _attn_body(be_ref, hs_ref, cs_ref, pos_ref, ln1_ref,
               wq_ref, wo_ref, pq_ref, k_ref, v_ref, y1_ref, o_ref):
    hs = hs_ref[...]
    x = _rms(hs) * ln1_ref[0]
    q = jnp.dot(x, wq_ref[0], preferred_element_type=jnp.float32)
    cfull = jnp.concatenate([cs_ref[:, :DH]] * H, axis=1)
    sfull = jnp.concatenate([cs_ref[:, DH:]] * H, axis=1)
    q = q * cfull + jnp.dot(
        q, pq_ref[...], preferred_element_type=jnp.float32) * sfull
    q = q * SCALE
    pos_q = pos_ref[...]
    pos_c = pos_q[:, :1]
    maxpos = jnp.max(jnp.minimum(pos_q, S - 1))

    def attn_width(W):
        def go():
            kiota = jax.lax.broadcasted_iota(jnp.int32, (BS, W), 1)
            mask = pos_c >= kiota
            for hh in range(H):
                kv = hh // (H // HKV)
                qh = q[:, hh * DH:(hh + 1) * DH]
                kh = k_ref[0, :W, kv * DH:(kv + 1) * DH]
                vh = v_ref[0, :W, kv * DH:(kv + 1) * DH]
                s = jax.lax.dot_general(qh, kh, (((1,), (1,)), ((), ())),
                                        preferred_element_type=jnp.float32)
                s = jnp.where(mask, s, -1e30)
                m = jnp.max(s, axis=-1, keepdims=True)
                p = jnp.exp(s - m)
                p = p / jnp.sum(p, axis=-1, keepdims=True)
                o_ref[:, hh * DH:(hh + 1) * DH] = jnp.dot(
                    p, vh, preferred_element_type=jnp.float32)
        return go

    small = maxpos < (S // 2)
    pl.when(small)(attn_width(S // 2))
    pl.when(jnp.logical_not(small))(attn_width(S))
    y1_ref[...] = hs + jnp.dot(o_ref[...], wo_ref[0],
                               preferred_element_type=jnp.float32)


def _attn_sparse(block_expert, hs, cs_g, pos_col, ln1, Wq, Wo, Pq,
                 Kc, Vc):
    grid_spec = pltpu.PrefetchScalarGridSpec(
        num_scalar_prefetch=1,
        grid=(NBLK,),
        in_specs=[
            pl.BlockSpec((BS, D), lambda b, be: (b, 0)),
            pl.BlockSpec((BS, 2 * DH), lambda b, be: (b, 0)),
            pl.BlockSpec((BS, 128), lambda b, be: (b, 0)),
            pl.BlockSpec((1, 1, D), lambda b, be: (be[b], 0, 0)),
            pl.BlockSpec((1, D, H * DH), lambda b, be: (be[b], 0, 0)),
            pl.BlockSpec((1, H * DH, D), lambda b, be: (be[b], 0, 0)),
            pl.BlockSpec((H * DH, H * DH), lambda b, be: (0, 0)),
            pl.BlockSpec((1, S, HKV * DH), lambda b, be: (be[b], 0, 0)),
            pl.BlockSpec((1, S, HKV * DH), lambda b, be: (be[b], 0, 0)),
        ],
        out_specs=pl.BlockSpec((BS, D), lambda b, be: (b, 0)),
        scratch_shapes=[pltpu.VMEM((BS, H * DH), jnp.float32)],
    )
    return pl.pallas_call(
        _attn_body,
        grid_spec=grid_spec,
        out_shape=jax.ShapeDtypeStruct((NPAD, D), jnp.float32),
    )(block_expert, hs, cs_g, pos_col, ln1, Wq, Wo, Pq, Kc, Vc)


def _mlp_body(be_ref, y1_ref, ln2_ref, wg_ref, wu_ref, wd_ref, y2_ref):
    a = y1_ref[...]
    x2 = _rms(a) * ln2_ref[0]
    g = jnp.dot(x2, wg_ref[0], preferred_element_type=jnp.float32)
    u = jnp.dot(x2, wu_ref[0], preferred_element_type=jnp.float32)
    act = (g / (1.0 + jnp.exp(-g))) * u
    y2_ref[...] = a + jnp.dot(act, wd_ref[0],
                              preferred_element_type=jnp.float32)


def _mlp_sparse(block_expert, y1, ln2, Wgate, Wup, Wdown):
    grid_spec = pltpu.PrefetchScalarGridSpec(
        num_scalar_prefetch=1,
        grid=(NBLK,),
        in_specs=[
            pl.BlockSpec((BS, D), lambda b, be: (b, 0)),
            pl.BlockSpec((1, 1, D), lambda b, be: (be[b], 0, 0)),
            pl.BlockSpec((1, D, FF), lambda b, be: (be[b], 0, 0)),
            pl.BlockSpec((1, D, FF), lambda b, be: (be[b], 0, 0)),
            pl.BlockSpec((1, FF, D), lambda b, be: (be[b], 0, 0)),
        ],
        out_specs=pl.BlockSpec((BS, D), lambda b, be: (b, 0)),
    )
    return pl.pallas_call(
        _mlp_body,
        grid_spec=grid_spec,
        out_shape=jax.ShapeDtypeStruct((NPAD, D), jnp.float32),
    )(block_expert, y1, ln2, Wgate, Wup, Wdown)


def _combine_body(g0_ref, g1_ref, w0_ref, w1_ref, out_ref):
    out_ref[...] = (g0_ref[...] * w0_ref[:, :1]
                    + g1_ref[...] * w1_ref[:, :1])


def _combine(g0, g1, w0c, w1c):
    return pl.pallas_call(
        _combine_body,
        grid=(S // SB,),
        in_specs=[
            pl.BlockSpec((SB, D), lambda i: (i, 0)),
            pl.BlockSpec((SB, D), lambda i: (i, 0)),
            pl.BlockSpec((SB, 128), lambda i: (i, 0)),
            pl.BlockSpec((SB, 128), lambda i: (i, 0)),
        ],
        out_specs=pl.BlockSpec((SB, D), lambda i: (i, 0)),
        out_shape=jax.ShapeDtypeStruct((S, D), jnp.float32),
    )(g0, g1, w0c, w1c)


@jax.jit
def kernel(hidden_states, Wg1, Wg2, ln1, ln2, Wq, Wk, Wv, Wo,
           Wgate, Wup, Wdown, position_ids):
    h2d = hidden_states[0]
    cos_t = jnp.asarray(_COS_NP)
    sin_t = jnp.asarray(_SIN_NP)
    cos4 = jnp.asarray(_COS4_NP)
    sin4 = jnp.asarray(_SIN4_NP)
    Pq = jnp.asarray(_PQ_NP)
    Pk = jnp.asarray(_PK_NP)

    router_logits, w2, sel2, rank2, countsf = _router(h2d, Wg1, Wg2)

    counts = countsf[0].astype(jnp.int32)
    padded = ((counts + BS - 1) // BS) * BS
    cum_pad = jnp.cumsum(padded)
    pstart = (cum_pad - padded).astype(jnp.int32)
    block_expert = jnp.minimum(
        jnp.searchsorted(cum_pad,
                         jnp.arange(NBLK, dtype=jnp.int32) * BS,
                         side='right'),
        E - 1).astype(jnp.int32)

    pstart16 = jnp.concatenate(
        [pstart, jnp.zeros((16 - E,), jnp.int32)])[None, :]
    dest2 = _dest(sel2, rank2, pstart16)
    d0 = dest2[:, 0]
    d1 = dest2[:, 1]
    tok_sorted = _scatter_sc(dest2.reshape(-1))
    cossin = jnp.concatenate([cos_t, sin_t], axis=1)
    hs, cs_g = _gather_sc(tok_sorted, h2d, cossin)
    pos_col = jnp.broadcast_to(tok_sorted[:, None], (NPAD, 128))

    Kc, Vc = _kv_dense(h2d, ln1[:, None, :], Wk, Wv, cos4, sin4, Pk)
    y1 = _attn_sparse(block_expert, hs, cs_g, pos_col,
                      ln1[:, None, :], Wq, Wo, Pq, Kc, Vc)
    y2 = _mlp_sparse(block_expert, y1, ln2[:, None, :], Wgate, Wup, Wdown)

    g0, g1 = _combine_gather_sc(y2, d0, d1)
    w0c = jnp.broadcast_to(w2[:, :1], (S, 128))
    w1c = jnp.broadcast_to(w2[:, 1:2], (S, 128))
    final = _combine(g0, g1, w0c, w1c)

    return final[None], router_logits[None]


# speedup vs baseline: 1.4348x; 1.0066x over previous
import functools
import numpy as np
import jax
import jax.numpy as jnp
from jax import lax
from jax.experimental import pallas as pl
from jax.experimental.pallas import tpu as pltpu
from jax.experimental.pallas import tpu_sc as plsc

B, S, D = 1, 2048, 768
H, HKV, DH = 12, 4, 64
E, TOPK = 8, 2
FF = 3072
NA = S * TOPK
BS = 128
NPAD = NA + E * BS
NBLK = NPAD // BS
SB = 256
EPS = 1e-6
SCALE = 1.0 / float(np.sqrt(DH))

NC, NW = 2, 32
TPW = S // NW
SPW = NPAD // NW


def _rope_tables_np():
    inv = 1.0 / (10000.0 ** (np.arange(0, DH, 2, dtype=np.float64) / DH))
    t = np.arange(S, dtype=np.float64)
    freqs = np.outer(t, inv)
    emb = np.concatenate([freqs, freqs], axis=-1)
    return np.cos(emb).astype(np.float32), np.sin(emb).astype(np.float32)


def _rot_perm_np(width):
    n = width // DH
    P = np.zeros((width, width), dtype=np.float32)
    half = DH // 2
    for c in range(n):
        b = c * DH
        for i in range(half):
            P[b + half + i, b + i] = -1.0
            P[b + i, b + half + i] = 1.0
    return P

_COS_NP, _SIN_NP = _rope_tables_np()
_COS4_NP = np.tile(_COS_NP, (1, HKV))
_SIN4_NP = np.tile(_SIN_NP, (1, HKV))
_PQ_NP = _rot_perm_np(H * DH)
_PK_NP = _rot_perm_np(HKV * DH)


def _rms(x, eps=EPS):
    v = jnp.mean(x * x, axis=-1, keepdims=True)
    return x * jax.lax.rsqrt(v + eps)


def _router_body(h_ref, wg1_ref, wg2_ref, logits_ref, w2_ref, sel2_ref,
                 rank2_ref, counts_ref, carry_ref):
    i = pl.program_id(0)

    @pl.when(i == 0)
    def _():
        carry_ref[...] = jnp.zeros((1, E), jnp.float32)

    x = h_ref[...]
    t = jnp.dot(x, wg1_ref[...], preferred_element_type=jnp.float32)
    logits = jnp.dot(t, wg2_ref[...], preferred_element_type=jnp.float32)
    logits_ref[...] = logits
    m = jnp.max(logits, axis=-1, keepdims=True)
    p = jnp.exp(logits - m)
    rw = p / jnp.sum(p, axis=-1, keepdims=True)
    iota = jax.lax.broadcasted_iota(jnp.int32, rw.shape, 1)
    m0 = jnp.max(rw, axis=-1, keepdims=True)
    sel0 = jnp.min(jnp.where(rw >= m0, iota, E), axis=-1, keepdims=True)
    oh0 = (iota == sel0).astype(jnp.float32)
    rw2 = jnp.where(iota == sel0, -1.0, rw)
    m1 = jnp.max(rw2, axis=-1, keepdims=True)
    sel1 = jnp.min(jnp.where(rw2 >= m1, iota, E), axis=-1, keepdims=True)
    oh1 = (iota == sel1).astype(jnp.float32)
    den = m0 + m1 + 1e-9
    w2_ref[...] = jnp.concatenate([m0 / den, m1 / den], axis=-1)
    sel2_ref[...] = jnp.concatenate([sel0, sel1], axis=-1)

    ri = jax.lax.broadcasted_iota(jnp.int32, (SB, SB), 0)
    ci = jax.lax.broadcasted_iota(jnp.int32, (SB, SB), 1)
    tri = (ci < ri).astype(jnp.float32)
    both = oh0 + oh1
    cex = jnp.dot(tri, both, preferred_element_type=jnp.float32)
    carry = carry_ref[...]
    r0 = jnp.sum((carry + cex) * oh0, axis=-1, keepdims=True)
    r1 = jnp.sum((carry + cex) * oh1, axis=-1, keepdims=True)
    rank2_ref[...] = jnp.concatenate([r0, r1], axis=-1).astype(jnp.int32)
    carry = carry + jnp.sum(both, axis=0, keepdims=True)
    carry_ref[...] = carry
    counts_ref[...] = carry


def _router(h2d, Wg1, Wg2):
    return pl.pallas_call(
        _router_body,
        grid=(S // SB,),
        in_specs=[
            pl.BlockSpec((SB, D), lambda i: (i, 0)),
            pl.BlockSpec((D, D), lambda i: (0, 0)),
            pl.BlockSpec((D, E), lambda i: (0, 0)),
        ],
        out_specs=[
            pl.BlockSpec((SB, E), lambda i: (i, 0)),
            pl.BlockSpec((SB, TOPK), lambda i: (i, 0)),
            pl.BlockSpec((SB, TOPK), lambda i: (i, 0)),
            pl.BlockSpec((SB, TOPK), lambda i: (i, 0)),
            pl.BlockSpec((1, E), lambda i: (0, 0)),
        ],
        out_shape=[
            jax.ShapeDtypeStruct((S, E), jnp.float32),
            jax.ShapeDtypeStruct((S, TOPK), jnp.float32),
            jax.ShapeDtypeStruct((S, TOPK), jnp.int32),
            jax.ShapeDtypeStruct((S, TOPK), jnp.int32),
            jax.ShapeDtypeStruct((1, E), jnp.float32),
        ],
        scratch_shapes=[pltpu.VMEM((1, E), jnp.float32)],
    )(h2d, Wg1, Wg2)


def _dest_body(sel2_ref, rank2_ref, pst_ref, dest2_ref):
    sel = sel2_ref[...]
    acc = rank2_ref[...]
    for e in range(E):
        acc = acc + jnp.where(sel == e, pst_ref[0, e], 0)
    dest2_ref[...] = acc


def _dest(sel2, rank2, pstart16):
    return pl.pallas_call(
        _dest_body,
        grid=(1,),
        in_specs=[
            pl.BlockSpec((S, TOPK), lambda i: (0, 0)),
            pl.BlockSpec((S, TOPK), lambda i: (0, 0)),
            pl.BlockSpec(memory_space=pltpu.SMEM),
        ],
        out_specs=pl.BlockSpec((S, TOPK), lambda i: (0, 0)),
        out_shape=jax.ShapeDtypeStruct((S, TOPK), jnp.int32),
    )(sel2, rank2, pstart16)


def _scatter_sc(sel_flat, rank_flat, pstart_rep):
    mesh = plsc.VectorSubcoreMesh(core_axis_name="c", subcore_axis_name="s")
    APW = NA // NW

    @functools.partial(
        pl.kernel,
        out_type=[
            jax.ShapeDtypeStruct((NPAD,), jnp.int32),
            jax.ShapeDtypeStruct((NA,), jnp.int32),
        ],
        mesh=mesh,
        scratch_types=[
            pltpu.VMEM((APW,), jnp.int32),
            pltpu.VMEM((APW,), jnp.int32),
            pltpu.VMEM((APW,), jnp.int32),
            pltpu.VMEM((E, 16), jnp.int32),
            pltpu.SemaphoreType.DMA,
        ],
    )
    def k(sel_h, rank_h, pst_h, tok_h, dest_h, sel_v, idx_v, val_v,
          pst_v, sem):
        wid = lax.axis_index("s") * NC + lax.axis_index("c")
        base = wid * APW
        asl = pl.ds(base, APW)
        pltpu.sync_copy(pst_h, pst_v)
        pltpu.sync_copy(sel_h.at[asl], sel_v)
        pltpu.sync_copy(rank_h.at[asl], idx_v)
        iota16 = jax.lax.broadcasted_iota(jnp.int32, (16,), 0)

        def vbody(i, _):
            sl = pl.ds(i * 16, 16)
            ent = base + i * 16 + iota16
            val_v[sl] = lax.shift_right_logical(ent, 1)
            sel16 = sel_v[sl]
            d = idx_v[sl]
            for e in range(E):
                d = d + jnp.where(sel16 == e, pst_v[e, :], 0)
            idx_v[sl] = d
            return 0

        lax.fori_loop(0, APW // 16, vbody, 0)
        pltpu.sync_copy(idx_v, dest_h.at[asl])
        pltpu.async_copy(val_v, tok_h.at[idx_v], sem).wait()

    return k(sel_flat, rank_flat, pstart_rep)


_GC = 5
_GR = SPW // _GC


def _gather_sc(tok_sorted, h2d, cossin):
    mesh = plsc.VectorSubcoreMesh(core_axis_name="c", subcore_axis_name="s")

    @functools.partial(
        pl.kernel,
        out_type=[
            jax.ShapeDtypeStruct((NPAD, D), jnp.float32),
            jax.ShapeDtypeStruct((NPAD, 2 * DH), jnp.float32),
        ],
        mesh=mesh,
        scratch_types=[
            pltpu.VMEM((2, _GR), jnp.int32),
            pltpu.VMEM((2, _GR, D), jnp.float32),
            pltpu.VMEM((2, _GR, 2 * DH), jnp.float32),
            pltpu.SemaphoreType.DMA,
            pltpu.SemaphoreType.DMA,
        ],
    )
    def k(tok_h, h_h, cs_h, hs_h, csg_h, idx_v, hbuf, cbuf, sem0, sem1):
        wid = lax.axis_index("s") * NC + lax.axis_index("c")
        base = wid * SPW
        sems = (sem0, sem1)

        def start(c):
            b = c & 1
            pltpu.sync_copy(tok_h.at[pl.ds(base + c * _GR, _GR)],
                            idx_v.at[b])

            def clamp(i, _):
                sl = pl.ds(i * 16, 16)
                idx_v[b, sl] = jnp.minimum(
                    jnp.maximum(idx_v[b, sl], 0), S - 1)
                return 0

            lax.fori_loop(0, _GR // 16, clamp, 0)
            hcp = pltpu.async_copy(h_h.at[idx_v.at[b]], hbuf.at[b], sems[b])
            ccp = pltpu.async_copy(cs_h.at[idx_v.at[b]], cbuf.at[b], sems[b])
            return hcp, ccp

        def drain(c, cps):
            b = c & 1
            rs = pl.ds(base + c * _GR, _GR)
            cps[0].wait()
            cps[1].wait()
            pltpu.sync_copy(hbuf.at[b], hs_h.at[rs])
            pltpu.sync_copy(cbuf.at[b], csg_h.at[rs])

        prev = start(0)
        for c in range(1, _GC):
            cur = start(c)
            drain(c - 1, prev)
            prev = cur
        drain(_GC - 1, prev)

    return k(tok_sorted, h2d, cossin)


_CC = 2
_CR = TPW // _CC


def _combine_gather_sc(y2, d0, d1, w0c, w1c):
    mesh = plsc.VectorSubcoreMesh(core_axis_name="c", subcore_axis_name="s")

    @functools.partial(
        pl.kernel,
        out_type=jax.ShapeDtypeStruct((S, D), jnp.float32),
        mesh=mesh,
        scratch_types=[
            pltpu.VMEM((_CR,), jnp.int32),
            pltpu.VMEM((_CR, D), jnp.float32),
            pltpu.VMEM((_CR, D), jnp.float32),
            pltpu.VMEM((_CR, 128), jnp.float32),
            pltpu.VMEM((_CR, 128), jnp.float32),
            pltpu.SemaphoreType.DMA,
        ],
    )
    def k(y2_h, d0_h, d1_h, w0_h, w1_h, out_h, idx_v, b0, b1, w0_v, w1_v,
          sem):
        wid = lax.axis_index("s") * NC + lax.axis_index("c")
        base = wid * TPW
        for c in range(_CC):
            rs = pl.ds(base + c * _CR, _CR)
            pltpu.sync_copy(d0_h.at[rs], idx_v)
            pltpu.async_copy(y2_h.at[idx_v], b0, sem).wait()
            pltpu.sync_copy(d1_h.at[rs], idx_v)
            pltpu.async_copy(y2_h.at[idx_v], b1, sem).wait()
            pltpu.sync_copy(w0_h.at[rs], w0_v)
            pltpu.sync_copy(w1_h.at[rs], w1_v)

            def rbody(r, _):
                w0 = w0_v[r, pl.ds(0, 16)]
                w1 = w1_v[r, pl.ds(0, 16)]
                for j in range(D // 16):
                    js = pl.ds(j * 16, 16)
                    b0[r, js] = b0[r, js] * w0 + b1[r, js] * w1
                return 0

            lax.fori_loop(0, _CR, rbody, 0)
            pltpu.sync_copy(b0, out_h.at[rs])

    return k(y2, d0, d1, w0c, w1c)


def _kv_body(h_ref, ln1_ref, wk_ref, wv_ref, cos_ref, sin_ref, pk_ref,
             k_ref, v_ref):
    x = _rms(h_ref[...]) * ln1_ref[0]
    k = jnp.dot(x, wk_ref[0], preferred_element_type=jnp.float32)
    k = k * cos_ref[...] + jnp.dot(
        k, pk_ref[...], preferred_element_type=jnp.float32) * sin_ref[...]
    k_ref[0] = k
    v_ref[0] = jnp.dot(x, wv_ref[0], preferred_element_type=jnp.float32)


def _kv_dense(h2d, ln1, Wk, Wv, cos4, sin4, Pk):
    return pl.pallas_call(
        _kv_body,
        grid=(E, S // SB),
        in_specs=[
            pl.BlockSpec((SB, D), lambda e, s: (s, 0)),
            pl.BlockSpec((1, 1, D), lambda e, s: (e, 0, 0)),
            pl.BlockSpec((1, D, HKV * DH), lambda e, s: (e, 0, 0)),
            pl.BlockSpec((1, D, HKV * DH), lambda e, s: (e, 0, 0)),
            pl.BlockSpec((SB, HKV * DH), lambda e, s: (s, 0)),
            pl.BlockSpec((SB, HKV * DH), lambda e, s: (s, 0)),
            pl.BlockSpec((HKV * DH, HKV * DH), lambda e, s: (0, 0)),
        ],
        out_specs=[
            pl.BlockSpec((1, SB, HKV * DH), lambda e, s: (e, s, 0)),
            pl.BlockSpec((1, SB, HKV * DH), lambda e, s: (e, s, 0)),
        ],
        out_shape=[
            jax.ShapeDtypeStruct((E, S, HKV * DH), jnp.float32),
            jax.ShapeDtypeStruct((E, S, HKV * DH), jnp.float32),
        ],
    )(h2d, ln1, Wk, Wv, cos4, sin4, Pk)


def _attn_body(be_ref, hs_ref, cs_ref, pos_ref, ln1_ref,
               wq_ref, wo_ref, pq_ref, k_ref, v_ref, y1_ref, o_ref):
    hs = hs_ref[...]
    x = _rms(hs) * ln1_ref[0]
    q = jnp.dot(x, wq_ref[0], preferred_element_type=jnp.float32)
    cfull = jnp.concatenate([cs_ref[:, :DH]] * H, axis=1)
    sfull = jnp.concatenate([cs_ref[:, DH:]] * H, axis=1)
    q = q * cfull + jnp.dot(
        q, pq_ref[...], preferred_element_type=jnp.float32) * sfull
    q = q * SCALE
    pos_q = pos_ref[...]
    pos_c = pos_q[:, :1]
    maxpos = jnp.max(jnp.minimum(pos_q, S - 1))

    def attn_width(W):
        def go():
            kiota = jax.lax.broadcasted_iota(jnp.int32, (BS, W), 1)
            mask = pos_c >= kiota
            for hh in range(H):
                kv = hh // (H // HKV)
                qh = q[:, hh * DH:(hh + 1) * DH]
                kh = k_ref[0, :W, kv * DH:(kv + 1) * DH]
                vh = v_ref[0, :W, kv * DH:(kv + 1) * DH]
                s = jax.lax.dot_general(qh, kh, (((1,), (1,)), ((), ())),
                                        preferred_element_type=jnp.float32)
                s = jnp.where(mask, s, -1e30)
                m = jnp.max(s, axis=-1, keepdims=True)
                p = jnp.exp(s - m)
                p = p / jnp.sum(p, axis=-1, keepdims=True)
                o_ref[:, hh * DH:(hh + 1) * DH] = jnp.dot(
                    p, vh, preferred_element_type=jnp.float32)
        return go

    small = maxpos < (S // 2)
    pl.when(small)(attn_width(S // 2))
    pl.when(jnp.logical_not(small))(attn_width(S))
    y1_ref[...] = hs + jnp.dot(o_ref[...], wo_ref[0],
                               preferred_element_type=jnp.float32)


def _attn_sparse(block_expert, hs, cs_g, pos_col, ln1, Wq, Wo, Pq,
                 Kc, Vc):
    grid_spec = pltpu.PrefetchScalarGridSpec(
        num_scalar_prefetch=1,
        grid=(NBLK,),
        in_specs=[
            pl.BlockSpec((BS, D), lambda b, be: (b, 0)),
            pl.BlockSpec((BS, 2 * DH), lambda b, be: (b, 0)),
            pl.BlockSpec((BS, 128), lambda b, be: (b, 0)),
            pl.BlockSpec((1, 1, D), lambda b, be: (be[b], 0, 0)),
            pl.BlockSpec((1, D, H * DH), lambda b, be: (be[b], 0, 0)),
            pl.BlockSpec((1, H * DH, D), lambda b, be: (be[b], 0, 0)),
            pl.BlockSpec((H * DH, H * DH), lambda b, be: (0, 0)),
            pl.BlockSpec((1, S, HKV * DH), lambda b, be: (be[b], 0, 0)),
            pl.BlockSpec((1, S, HKV * DH), lambda b, be: (be[b], 0, 0)),
        ],
        out_specs=pl.BlockSpec((BS, D), lambda b, be: (b, 0)),
        scratch_shapes=[pltpu.VMEM((BS, H * DH), jnp.float32)],
    )
    return pl.pallas_call(
        _attn_body,
        grid_spec=grid_spec,
        out_shape=jax.ShapeDtypeStruct((NPAD, D), jnp.float32),
    )(block_expert, hs, cs_g, pos_col, ln1, Wq, Wo, Pq, Kc, Vc)


def _mlp_body(be_ref, y1_ref, ln2_ref, wg_ref, wu_ref, wd_ref, y2_ref):
    a = y1_ref[...]
    x2 = _rms(a) * ln2_ref[0]
    g = jnp.dot(x2, wg_ref[0], preferred_element_type=jnp.float32)
    u = jnp.dot(x2, wu_ref[0], preferred_element_type=jnp.float32)
    act = (g / (1.0 + jnp.exp(-g))) * u
    y2_ref[...] = a + jnp.dot(act, wd_ref[0],
                              preferred_element_type=jnp.float32)


def _mlp_sparse(block_expert, y1, ln2, Wgate, Wup, Wdown):
    grid_spec = pltpu.PrefetchScalarGridSpec(
        num_scalar_prefetch=1,
        grid=(NBLK,),
        in_specs=[
            pl.BlockSpec((BS, D), lambda b, be: (b, 0)),
            pl.BlockSpec((1, 1, D), lambda b, be: (be[b], 0, 0)),
            pl.BlockSpec((1, D, FF), lambda b, be: (be[b], 0, 0)),
            pl.BlockSpec((1, D, FF), lambda b, be: (be[b], 0, 0)),
            pl.BlockSpec((1, FF, D), lambda b, be: (be[b], 0, 0)),
        ],
        out_specs=pl.BlockSpec((BS, D), lambda b, be: (b, 0)),
    )
    return pl.pallas_call(
        _mlp_body,
        grid_spec=grid_spec,
        out_shape=jax.ShapeDtypeStruct((NPAD, D), jnp.float32),
    )(block_expert, y1, ln2, Wgate, Wup, Wdown)


def _combine_body(g0_ref, g1_ref, w0_ref, w1_ref, out_ref):
    out_ref[...] = (g0_ref[...] * w0_ref[:, :1]
                    + g1_ref[...] * w1_ref[:, :1])


def _combine(g0, g1, w0c, w1c):
    return pl.pallas_call(
        _combine_body,
        grid=(S // SB,),
        in_specs=[
            pl.BlockSpec((SB, D), lambda i: (i, 0)),
            pl.BlockSpec((SB, D), lambda i: (i, 0)),
            pl.BlockSpec((SB, 128), lambda i: (i, 0)),
            pl.BlockSpec((SB, 128), lambda i: (i, 0)),
        ],
        out_specs=pl.BlockSpec((SB, D), lambda i: (i, 0)),
        out_shape=jax.ShapeDtypeStruct((S, D), jnp.float32),
    )(g0, g1, w0c, w1c)


@jax.jit
def kernel(hidden_states, Wg1, Wg2, ln1, ln2, Wq, Wk, Wv, Wo,
           Wgate, Wup, Wdown, position_ids):
    h2d = hidden_states[0]
    cos_t = jnp.asarray(_COS_NP)
    sin_t = jnp.asarray(_SIN_NP)
    cos4 = jnp.asarray(_COS4_NP)
    sin4 = jnp.asarray(_SIN4_NP)
    Pq = jnp.asarray(_PQ_NP)
    Pk = jnp.asarray(_PK_NP)

    router_logits, w2, sel2, rank2, countsf = _router(h2d, Wg1, Wg2)

    counts = countsf[0].astype(jnp.int32)
    padded = ((counts + BS - 1) // BS) * BS
    cum_pad = jnp.cumsum(padded)
    pstart = (cum_pad - padded).astype(jnp.int32)
    block_expert = jnp.minimum(
        jnp.searchsorted(cum_pad,
                         jnp.arange(NBLK, dtype=jnp.int32) * BS,
                         side='right'),
        E - 1).astype(jnp.int32)

    pstart_rep = jnp.broadcast_to(pstart[:, None], (E, 16))
    tok_sorted, dest_flat = _scatter_sc(
        sel2.reshape(-1), rank2.reshape(-1), pstart_rep)
    dest2 = dest_flat.reshape(S, TOPK)
    d0 = dest2[:, 0]
    d1 = dest2[:, 1]
    cossin = jnp.concatenate([cos_t, sin_t], axis=1)
    hs, cs_g = _gather_sc(tok_sorted, h2d, cossin)
    pos_col = jnp.broadcast_to(tok_sorted[:, None], (NPAD, 128))

    Kc, Vc = _kv_dense(h2d, ln1[:, None, :], Wk, Wv, cos4, sin4, Pk)
    y1 = _attn_sparse(block_expert, hs, cs_g, pos_col,
                      ln1[:, None, :], Wq, Wo, Pq, Kc, Vc)
    y2 = _mlp_sparse(block_expert, y1, ln2[:, None, :], Wgate, Wup, Wdown)

    w0c = jnp.broadcast_to(w2[:, :1], (S, 128))
    w1c = jnp.broadcast_to(w2[:, 1:2], (S, 128))
    final = _combine_gather_sc(y2, d0, d1, w0c, w1c)

    return final[None], router_logits[None]


# speedup vs baseline: 1.4670x; 1.0224x over previous
import functools
import numpy as np
import jax
import jax.numpy as jnp
from jax import lax
from jax.experimental import pallas as pl
from jax.experimental.pallas import tpu as pltpu
from jax.experimental.pallas import tpu_sc as plsc

B, S, D = 1, 2048, 768
H, HKV, DH = 12, 4, 64
E, TOPK = 8, 2
FF = 3072
NA = S * TOPK
BS = 128
NPAD = NA + E * BS
NBLK = NPAD // BS
SB = 256
EPS = 1e-6
SCALE = 1.0 / float(np.sqrt(DH))

NC, NW = 2, 32
TPW = S // NW
SPW = NPAD // NW


def _rope_tables_np():
    inv = 1.0 / (10000.0 ** (np.arange(0, DH, 2, dtype=np.float64) / DH))
    t = np.arange(S, dtype=np.float64)
    freqs = np.outer(t, inv)
    emb = np.concatenate([freqs, freqs], axis=-1)
    return np.cos(emb).astype(np.float32), np.sin(emb).astype(np.float32)


def _rot_perm_np(width):
    n = width // DH
    P = np.zeros((width, width), dtype=np.float32)
    half = DH // 2
    for c in range(n):
        b = c * DH
        for i in range(half):
            P[b + half + i, b + i] = -1.0
            P[b + i, b + half + i] = 1.0
    return P

_COS_NP, _SIN_NP = _rope_tables_np()
_COS4_NP = np.tile(_COS_NP, (1, HKV))
_SIN4_NP = np.tile(_SIN_NP, (1, HKV))
_PQ_NP = _rot_perm_np(H * DH)
_PK_NP = _rot_perm_np(HKV * DH)


def _rms(x, eps=EPS):
    v = jnp.mean(x * x, axis=-1, keepdims=True)
    return x * jax.lax.rsqrt(v + eps)


def _router_body(h_ref, wg1_ref, wg2_ref, logits_ref, w2_ref, sel2_ref,
                 rank2_ref, counts_ref, carry_ref):
    i = pl.program_id(0)

    @pl.when(i == 0)
    def _():
        carry_ref[...] = jnp.zeros((1, E), jnp.float32)

    x = h_ref[...]
    t = jnp.dot(x, wg1_ref[...], preferred_element_type=jnp.float32)
    logits = jnp.dot(t, wg2_ref[...], preferred_element_type=jnp.float32)
    logits_ref[...] = logits
    m = jnp.max(logits, axis=-1, keepdims=True)
    p = jnp.exp(logits - m)
    rw = p / jnp.sum(p, axis=-1, keepdims=True)
    iota = jax.lax.broadcasted_iota(jnp.int32, rw.shape, 1)
    m0 = jnp.max(rw, axis=-1, keepdims=True)
    sel0 = jnp.min(jnp.where(rw >= m0, iota, E), axis=-1, keepdims=True)
    oh0 = (iota == sel0).astype(jnp.float32)
    rw2 = jnp.where(iota == sel0, -1.0, rw)
    m1 = jnp.max(rw2, axis=-1, keepdims=True)
    sel1 = jnp.min(jnp.where(rw2 >= m1, iota, E), axis=-1, keepdims=True)
    oh1 = (iota == sel1).astype(jnp.float32)
    den = m0 + m1 + 1e-9
    w2_ref[...] = jnp.concatenate([m0 / den, m1 / den], axis=-1)
    sel2_ref[...] = jnp.concatenate([sel0, sel1], axis=-1)

    ri = jax.lax.broadcasted_iota(jnp.int32, (SB, SB), 0)
    ci = jax.lax.broadcasted_iota(jnp.int32, (SB, SB), 1)
    tri = (ci < ri).astype(jnp.float32)
    both = oh0 + oh1
    cex = jnp.dot(tri, both, preferred_element_type=jnp.float32)
    carry = carry_ref[...]
    r0 = jnp.sum((carry + cex) * oh0, axis=-1, keepdims=True)
    r1 = jnp.sum((carry + cex) * oh1, axis=-1, keepdims=True)
    rank2_ref[...] = jnp.concatenate([r0, r1], axis=-1).astype(jnp.int32)
    carry = carry + jnp.sum(both, axis=0, keepdims=True)
    carry_ref[...] = carry
    counts_ref[...] = carry


def _router(h2d, Wg1, Wg2):
    return pl.pallas_call(
        _router_body,
        grid=(S // SB,),
        in_specs=[
            pl.BlockSpec((SB, D), lambda i: (i, 0)),
            pl.BlockSpec((D, D), lambda i: (0, 0)),
            pl.BlockSpec((D, E), lambda i: (0, 0)),
        ],
        out_specs=[
            pl.BlockSpec((SB, E), lambda i: (i, 0)),
            pl.BlockSpec((SB, TOPK), lambda i: (i, 0)),
            pl.BlockSpec((SB, TOPK), lambda i: (i, 0)),
            pl.BlockSpec((SB, TOPK), lambda i: (i, 0)),
            pl.BlockSpec((1, E), lambda i: (0, 0)),
        ],
        out_shape=[
            jax.ShapeDtypeStruct((S, E), jnp.float32),
            jax.ShapeDtypeStruct((S, TOPK), jnp.float32),
            jax.ShapeDtypeStruct((S, TOPK), jnp.int32),
            jax.ShapeDtypeStruct((S, TOPK), jnp.int32),
            jax.ShapeDtypeStruct((1, E), jnp.float32),
        ],
        scratch_shapes=[pltpu.VMEM((1, E), jnp.float32)],
    )(h2d, Wg1, Wg2)


def _dest_body(sel2_ref, rank2_ref, pst_ref, dest2_ref):
    sel = sel2_ref[...]
    acc = rank2_ref[...]
    for e in range(E):
        acc = acc + jnp.where(sel == e, pst_ref[0, e], 0)
    dest2_ref[...] = acc


def _dest(sel2, rank2, pstart16):
    return pl.pallas_call(
        _dest_body,
        grid=(1,),
        in_specs=[
            pl.BlockSpec((S, TOPK), lambda i: (0, 0)),
            pl.BlockSpec((S, TOPK), lambda i: (0, 0)),
            pl.BlockSpec(memory_space=pltpu.SMEM),
        ],
        out_specs=pl.BlockSpec((S, TOPK), lambda i: (0, 0)),
        out_shape=jax.ShapeDtypeStruct((S, TOPK), jnp.int32),
    )(sel2, rank2, pstart16)


def _scatter_sc(sel_flat, rank_flat, pstart_rep):
    mesh = plsc.VectorSubcoreMesh(core_axis_name="c", subcore_axis_name="s")
    APW = NA // NW

    @functools.partial(
        pl.kernel,
        out_type=[
            jax.ShapeDtypeStruct((NPAD,), jnp.int32),
            jax.ShapeDtypeStruct((NA,), jnp.int32),
        ],
        mesh=mesh,
        scratch_types=[
            pltpu.VMEM((APW,), jnp.int32),
            pltpu.VMEM((APW,), jnp.int32),
            pltpu.VMEM((APW,), jnp.int32),
            pltpu.VMEM((E, 16), jnp.int32),
            pltpu.SemaphoreType.DMA,
        ],
    )
    def k(sel_h, rank_h, pst_h, tok_h, dest_h, sel_v, idx_v, val_v,
          pst_v, sem):
        wid = lax.axis_index("s") * NC + lax.axis_index("c")
        base = wid * APW
        asl = pl.ds(base, APW)
        pltpu.sync_copy(pst_h, pst_v)
        pltpu.sync_copy(sel_h.at[asl], sel_v)
        pltpu.sync_copy(rank_h.at[asl], idx_v)
        iota16 = jax.lax.broadcasted_iota(jnp.int32, (16,), 0)

        def vbody(i, _):
            sl = pl.ds(i * 16, 16)
            ent = base + i * 16 + iota16
            val_v[sl] = lax.shift_right_logical(ent, 1)
            sel16 = sel_v[sl]
            d = idx_v[sl]
            for e in range(E):
                d = d + jnp.where(sel16 == e, pst_v[e, :], 0)
            idx_v[sl] = d
            return 0

        lax.fori_loop(0, APW // 16, vbody, 0)
        pltpu.sync_copy(idx_v, dest_h.at[asl])
        pltpu.async_copy(val_v, tok_h.at[idx_v], sem).wait()

    return k(sel_flat, rank_flat, pstart_rep)


_GC = 5
_GR = SPW // _GC


def _gather_sc(tok_sorted, h2d, cossin):
    mesh = plsc.VectorSubcoreMesh(core_axis_name="c", subcore_axis_name="s")

    @functools.partial(
        pl.kernel,
        out_type=[
            jax.ShapeDtypeStruct((NPAD, D), jnp.float32),
            jax.ShapeDtypeStruct((NPAD, 2 * DH), jnp.float32),
        ],
        mesh=mesh,
        scratch_types=[
            pltpu.VMEM((2, _GR), jnp.int32),
            pltpu.VMEM((2, _GR, D), jnp.float32),
            pltpu.VMEM((2, _GR, 2 * DH), jnp.float32),
            pltpu.SemaphoreType.DMA,
            pltpu.SemaphoreType.DMA,
        ],
    )
    def k(tok_h, h_h, cs_h, hs_h, csg_h, idx_v, hbuf, cbuf, sem0, sem1):
        wid = lax.axis_index("s") * NC + lax.axis_index("c")
        base = wid * SPW
        sems = (sem0, sem1)

        def start(c):
            b = c & 1
            pltpu.sync_copy(tok_h.at[pl.ds(base + c * _GR, _GR)],
                            idx_v.at[b])

            def clamp(i, _):
                sl = pl.ds(i * 16, 16)
                idx_v[b, sl] = jnp.minimum(
                    jnp.maximum(idx_v[b, sl], 0), S - 1)
                return 0

            lax.fori_loop(0, _GR // 16, clamp, 0)
            hcp = pltpu.async_copy(h_h.at[idx_v.at[b]], hbuf.at[b], sems[b])
            ccp = pltpu.async_copy(cs_h.at[idx_v.at[b]], cbuf.at[b], sems[b])
            return hcp, ccp

        def drain(c, cps):
            b = c & 1
            rs = pl.ds(base + c * _GR, _GR)
            cps[0].wait()
            cps[1].wait()
            pltpu.sync_copy(hbuf.at[b], hs_h.at[rs])
            pltpu.sync_copy(cbuf.at[b], csg_h.at[rs])

        prev = start(0)
        for c in range(1, _GC):
            cur = start(c)
            drain(c - 1, prev)
            prev = cur
        drain(_GC - 1, prev)

    return k(tok_sorted, h2d, cossin)


_CC = 2
_CR = TPW // _CC


def _combine_gather_sc(y2, d0, d1, w0c, w1c):
    mesh = plsc.VectorSubcoreMesh(core_axis_name="c", subcore_axis_name="s")

    @functools.partial(
        pl.kernel,
        out_type=jax.ShapeDtypeStruct((S, D), jnp.float32),
        mesh=mesh,
        scratch_types=[
            pltpu.VMEM((_CR,), jnp.int32),
            pltpu.VMEM((_CR, D), jnp.float32),
            pltpu.VMEM((_CR, D), jnp.float32),
            pltpu.VMEM((_CR, 128), jnp.float32),
            pltpu.VMEM((_CR, 128), jnp.float32),
            pltpu.SemaphoreType.DMA,
        ],
    )
    def k(y2_h, d0_h, d1_h, w0_h, w1_h, out_h, idx_v, b0, b1, w0_v, w1_v,
          sem):
        wid = lax.axis_index("s") * NC + lax.axis_index("c")
        base = wid * TPW
        for c in range(_CC):
            rs = pl.ds(base + c * _CR, _CR)
            pltpu.sync_copy(d0_h.at[rs], idx_v)
            pltpu.async_copy(y2_h.at[idx_v], b0, sem).wait()
            pltpu.sync_copy(d1_h.at[rs], idx_v)
            pltpu.async_copy(y2_h.at[idx_v], b1, sem).wait()
            pltpu.sync_copy(w0_h.at[rs], w0_v)
            pltpu.sync_copy(w1_h.at[rs], w1_v)

            def rbody(r, _):
                w0 = w0_v[r, pl.ds(0, 16)]
                w1 = w1_v[r, pl.ds(0, 16)]
                for j in range(D // 16):
                    js = pl.ds(j * 16, 16)
                    b0[r, js] = b0[r, js] * w0 + b1[r, js] * w1
                return 0

            lax.fori_loop(0, _CR, rbody, 0)
            pltpu.sync_copy(b0, out_h.at[rs])

    return k(y2, d0, d1, w0c, w1c)


def _kv_body(h_ref, ln1_ref, wk_ref, wv_ref, cos_ref, sin_ref, pk_ref,
             k_ref, v_ref):
    x = _rms(h_ref[...]) * ln1_ref[0]
    k = jnp.dot(x, wk_ref[0], preferred_element_type=jnp.float32)
    k = k * cos_ref[...] + jnp.dot(
        k, pk_ref[...], preferred_element_type=jnp.float32) * sin_ref[...]
    k_ref[0] = k
    v_ref[0] = jnp.dot(x, wv_ref[0], preferred_element_type=jnp.float32)


def _kv_dense(h2d, ln1, Wk, Wv, cos4, sin4, Pk):
    return pl.pallas_call(
        _kv_body,
        grid=(E, S // SB),
        in_specs=[
            pl.BlockSpec((SB, D), lambda e, s: (s, 0)),
            pl.BlockSpec((1, 1, D), lambda e, s: (e, 0, 0)),
            pl.BlockSpec((1, D, HKV * DH), lambda e, s: (e, 0, 0)),
            pl.BlockSpec((1, D, HKV * DH), lambda e, s: (e, 0, 0)),
            pl.BlockSpec((SB, HKV * DH), lambda e, s: (s, 0)),
            pl.BlockSpec((SB, HKV * DH), lambda e, s: (s, 0)),
            pl.BlockSpec((HKV * DH, HKV * DH), lambda e, s: (0, 0)),
        ],
        out_specs=[
            pl.BlockSpec((1, SB, HKV * DH), lambda e, s: (e, s, 0)),
            pl.BlockSpec((1, SB, HKV * DH), lambda e, s: (e, s, 0)),
        ],
        out_shape=[
            jax.ShapeDtypeStruct((E, S, HKV * DH), jnp.float32),
            jax.ShapeDtypeStruct((E, S, HKV * DH), jnp.float32),
        ],
    )(h2d, ln1, Wk, Wv, cos4, sin4, Pk)


def _attn_body(be_ref, hs_ref, cs_ref, pos_ref, ln1_ref,
               wq_ref, wo_ref, pq_ref, k_ref, v_ref, y1_ref, o_ref):
    hs = hs_ref[...]
    x = _rms(hs) * ln1_ref[0]
    q = jnp.dot(x, wq_ref[0], preferred_element_type=jnp.float32)
    cfull = jnp.concatenate([cs_ref[:, :DH]] * H, axis=1)
    sfull = jnp.concatenate([cs_ref[:, DH:]] * H, axis=1)
    q = q * cfull + jnp.dot(
        q, pq_ref[...], preferred_element_type=jnp.float32) * sfull
    q = q * SCALE
    pos_q = pos_ref[...]
    pos_c = pos_q[:, :1]
    maxpos = jnp.max(jnp.minimum(pos_q, S - 1))

    def attn_width(W):
        def go():
            kiota = jax.lax.broadcasted_iota(jnp.int32, (BS, W), 1)
            mask = pos_c >= kiota
            for hh in range(H):
                kv = hh // (H // HKV)
                qh = q[:, hh * DH:(hh + 1) * DH]
                kh = k_ref[0, :W, kv * DH:(kv + 1) * DH]
                vh = v_ref[0, :W, kv * DH:(kv + 1) * DH]
                s = jax.lax.dot_general(qh, kh, (((1,), (1,)), ((), ())),
                                        preferred_element_type=jnp.float32)
                s = jnp.where(mask, s, -1e30)
                m = jnp.max(s, axis=-1, keepdims=True)
                p = jnp.exp(s - m)
                p = p / jnp.sum(p, axis=-1, keepdims=True)
                o_ref[:, hh * DH:(hh + 1) * DH] = jnp.dot(
                    p, vh, preferred_element_type=jnp.float32)
        return go

    b0 = maxpos < (S // 4)
    b1 = jnp.logical_and(maxpos >= (S // 4), maxpos < (S // 2))
    b2 = jnp.logical_and(maxpos >= (S // 2), maxpos < (3 * S // 4))
    b3 = maxpos >= (3 * S // 4)
    pl.when(b0)(attn_width(S // 4))
    pl.when(b1)(attn_width(S // 2))
    pl.when(b2)(attn_width(3 * S // 4))
    pl.when(b3)(attn_width(S))
    y1_ref[...] = hs + jnp.dot(o_ref[...], wo_ref[0],
                               preferred_element_type=jnp.float32)


def _attn_sparse(block_expert, hs, cs_g, pos_col, ln1, Wq, Wo, Pq,
                 Kc, Vc):
    grid_spec = pltpu.PrefetchScalarGridSpec(
        num_scalar_prefetch=1,
        grid=(NBLK,),
        in_specs=[
            pl.BlockSpec((BS, D), lambda b, be: (b, 0)),
            pl.BlockSpec((BS, 2 * DH), lambda b, be: (b, 0)),
            pl.BlockSpec((BS, 128), lambda b, be: (b, 0)),
            pl.BlockSpec((1, 1, D), lambda b, be: (be[b], 0, 0)),
            pl.BlockSpec((1, D, H * DH), lambda b, be: (be[b], 0, 0)),
            pl.BlockSpec((1, H * DH, D), lambda b, be: (be[b], 0, 0)),
            pl.BlockSpec((H * DH, H * DH), lambda b, be: (0, 0)),
            pl.BlockSpec((1, S, HKV * DH), lambda b, be: (be[b], 0, 0)),
            pl.BlockSpec((1, S, HKV * DH), lambda b, be: (be[b], 0, 0)),
        ],
        out_specs=pl.BlockSpec((BS, D), lambda b, be: (b, 0)),
        scratch_shapes=[pltpu.VMEM((BS, H * DH), jnp.float32)],
    )
    return pl.pallas_call(
        _attn_body,
        grid_spec=grid_spec,
        out_shape=jax.ShapeDtypeStruct((NPAD, D), jnp.float32),
    )(block_expert, hs, cs_g, pos_col, ln1, Wq, Wo, Pq, Kc, Vc)


def _mlp_body(be_ref, y1_ref, ln2_ref, wg_ref, wu_ref, wd_ref, y2_ref):
    a = y1_ref[...]
    x2 = _rms(a) * ln2_ref[0]
    g = jnp.dot(x2, wg_ref[0], preferred_element_type=jnp.float32)
    u = jnp.dot(x2, wu_ref[0], preferred_element_type=jnp.float32)
    act = (g / (1.0 + jnp.exp(-g))) * u
    y2_ref[...] = a + jnp.dot(act, wd_ref[0],
                              preferred_element_type=jnp.float32)


def _mlp_sparse(block_expert, y1, ln2, Wgate, Wup, Wdown):
    grid_spec = pltpu.PrefetchScalarGridSpec(
        num_scalar_prefetch=1,
        grid=(NBLK,),
        in_specs=[
            pl.BlockSpec((BS, D), lambda b, be: (b, 0)),
            pl.BlockSpec((1, 1, D), lambda b, be: (be[b], 0, 0)),
            pl.BlockSpec((1, D, FF), lambda b, be: (be[b], 0, 0)),
            pl.BlockSpec((1, D, FF), lambda b, be: (be[b], 0, 0)),
            pl.BlockSpec((1, FF, D), lambda b, be: (be[b], 0, 0)),
        ],
        out_specs=pl.BlockSpec((BS, D), lambda b, be: (b, 0)),
    )
    return pl.pallas_call(
        _mlp_body,
        grid_spec=grid_spec,
        out_shape=jax.ShapeDtypeStruct((NPAD, D), jnp.float32),
    )(block_expert, y1, ln2, Wgate, Wup, Wdown)


def _combine_body(g0_ref, g1_ref, w0_ref, w1_ref, out_ref):
    out_ref[...] = (g0_ref[...] * w0_ref[:, :1]
                    + g1_ref[...] * w1_ref[:, :1])


def _combine(g0, g1, w0c, w1c):
    return pl.pallas_call(
        _combine_body,
        grid=(S // SB,),
        in_specs=[
            pl.BlockSpec((SB, D), lambda i: (i, 0)),
            pl.BlockSpec((SB, D), lambda i: (i, 0)),
            pl.BlockSpec((SB, 128), lambda i: (i, 0)),
            pl.BlockSpec((SB, 128), lambda i: (i, 0)),
        ],
        out_specs=pl.BlockSpec((SB, D), lambda i: (i, 0)),
        out_shape=jax.ShapeDtypeStruct((S, D), jnp.float32),
    )(g0, g1, w0c, w1c)


@jax.jit
def kernel(hidden_states, Wg1, Wg2, ln1, ln2, Wq, Wk, Wv, Wo,
           Wgate, Wup, Wdown, position_ids):
    h2d = hidden_states[0]
    cos_t = jnp.asarray(_COS_NP)
    sin_t = jnp.asarray(_SIN_NP)
    cos4 = jnp.asarray(_COS4_NP)
    sin4 = jnp.asarray(_SIN4_NP)
    Pq = jnp.asarray(_PQ_NP)
    Pk = jnp.asarray(_PK_NP)

    router_logits, w2, sel2, rank2, countsf = _router(h2d, Wg1, Wg2)

    counts = countsf[0].astype(jnp.int32)
    padded = ((counts + BS - 1) // BS) * BS
    cum_pad = jnp.cumsum(padded)
    pstart = (cum_pad - padded).astype(jnp.int32)
    block_expert = jnp.minimum(
        jnp.searchsorted(cum_pad,
                         jnp.arange(NBLK, dtype=jnp.int32) * BS,
                         side='right'),
        E - 1).astype(jnp.int32)

    pstart_rep = jnp.broadcast_to(pstart[:, None], (E, 16))
    tok_sorted, dest_flat = _scatter_sc(
        sel2.reshape(-1), rank2.reshape(-1), pstart_rep)
    dest2 = dest_flat.reshape(S, TOPK)
    d0 = dest2[:, 0]
    d1 = dest2[:, 1]
    cossin = jnp.concatenate([cos_t, sin_t], axis=1)
    hs, cs_g = _gather_sc(tok_sorted, h2d, cossin)
    pos_col = jnp.broadcast_to(tok_sorted[:, None], (NPAD, 128))

    Kc, Vc = _kv_dense(h2d, ln1[:, None, :], Wk, Wv, cos4, sin4, Pk)
    y1 = _attn_sparse(block_expert, hs, cs_g, pos_col,
                      ln1[:, None, :], Wq, Wo, Pq, Kc, Vc)
    y2 = _mlp_sparse(block_expert, y1, ln2[:, None, :], Wgate, Wup, Wdown)

    w0c = jnp.broadcast_to(w2[:, :1], (S, 128))
    w1c = jnp.broadcast_to(w2[:, 1:2], (S, 128))
    final = _combine_gather_sc(y2, d0, d1, w0c, w1c)

    return final[None], router_logits[None]
